# baseline (speedup 1.0000x reference)
import functools

import numpy as np
import jax
from jax import lax
import jax.numpy as jnp
from jax.experimental import pallas as pl
from jax.experimental.pallas import tpu as pltpu
from jax.experimental.pallas import tpu_sc as plsc

B, S, D = 1, 2048, 1024
H, KV, HD = 16, 8, 64
E, TOPK, DI = 8, 2, 1024
EPS = 1e-05
THETA = 10000.0

BT = 256
NQ = S // BT
TILE = 128
NPAD = TOPK * S + E * TILE
NT = NPAD // TILE
NW = 32
TPW = S // NW

_INTERP = False


def _preattn_body(x_ref, w_ref, b_ref, rq_ref, cq_ref, sq_ref,
                  ln1_ref, q_out, k_out, v_out):
    x = x_ref[...]
    var = jnp.mean(x * x, axis=1, keepdims=True)
    h = (x * jax.lax.rsqrt(var + EPS) * ln1_ref[...]).astype(jnp.bfloat16)
    qkv = jnp.dot(h, w_ref[...], preferred_element_type=jnp.float32) + b_ref[...]
    hh = H * HD
    q = qkv[:, :hh]
    k = qkv[:, hh:2 * hh]
    v = qkv[:, 2 * hh:]
    cq, sq, rq = cq_ref[...], sq_ref[...], rq_ref[...]
    qr = jnp.dot(q.astype(jnp.bfloat16), rq, preferred_element_type=jnp.float32)
    kr = jnp.dot(k.astype(jnp.bfloat16), rq, preferred_element_type=jnp.float32)
    q_out[...] = ((q * cq + qr * sq) * 0.125).astype(jnp.bfloat16)
    k_out[...] = (k * cq + kr * sq).astype(jnp.bfloat16)
    v_out[...] = v.astype(jnp.bfloat16)


BK = 512


def _attn_body(q_ref, k_ref, v_ref, o_ref):
    iq = pl.program_id(1)
    q = q_ref[...]
    q2 = jnp.concatenate([q[:, :HD], q[:, HD:]], axis=0)
    row = jax.lax.broadcasted_iota(jnp.int32, (2 * BT, BK), 0)
    tok = iq * BT + jax.lax.rem(row, BT)
    col0 = jax.lax.broadcasted_iota(jnp.int32, (2 * BT, BK), 1)

    nchunks = iq // (BK // BT) + 1

    def body(kb, carry):
        acc, l = carry
        k_c = k_ref[pl.ds(kb * BK, BK), :HD]
        v_c = v_ref[pl.ds(kb * BK, BK), :HD]
        s = jax.lax.dot_general(q2, k_c, (((1,), (1,)), ((), ())),
                                preferred_element_type=jnp.float32)
        s = jax.lax.cond(
            kb == nchunks - 1,
            lambda t: jnp.where(col0 + kb * BK <= tok, t, -1e9),
            lambda t: t, s)
        e = jnp.exp(s)
        l = l + jnp.sum(e, axis=1, keepdims=True)
        acc = acc + jnp.dot(e.astype(jnp.bfloat16), v_c,
                            preferred_element_type=jnp.float32)
        return acc, l

    acc, l = jax.lax.fori_loop(
        0, nchunks, body,
        (jnp.zeros((2 * BT, HD), jnp.float32),
         jnp.zeros((2 * BT, 1), jnp.float32)))
    ctx2 = acc / l
    o_ref[...] = jnp.concatenate([ctx2[:BT], ctx2[BT:]], axis=1).astype(jnp.bfloat16)


def _postattn_body(ctx_ref, wo_ref, bo_ref, res_ref, ln2_ref, rw_ref,
                   x_out, h2_out, mi_out, mf_out, cnt_out, cnt_ref):
    i = pl.program_id(0)

    @pl.when(i == 0)
    def _():
        cnt_ref[...] = jnp.zeros((1, E), jnp.float32)

    xo = jnp.dot(ctx_ref[...], wo_ref[...], preferred_element_type=jnp.float32)
    x = res_ref[...] + xo + bo_ref[...]
    x_out[...] = x
    var = jnp.mean(x * x, axis=1, keepdims=True)
    h2 = x * jax.lax.rsqrt(var + EPS) * ln2_ref[...]
    h2_out[...] = h2
    logits = jnp.dot(h2, rw_ref[...], preferred_element_type=jnp.float32)
    lm = jnp.max(logits, axis=1, keepdims=True)
    ex = jnp.exp(logits - lm)
    p = ex / jnp.sum(ex, axis=1, keepdims=True)
    lane = jax.lax.broadcasted_iota(jnp.int32, (BT, E), 1)
    m1 = jnp.max(p, axis=1, keepdims=True)
    idx1 = jnp.min(jnp.where(p >= m1, lane, E), axis=1, keepdims=True)
    oh1 = (lane == idx1).astype(jnp.float32)
    p2 = jnp.where(lane == idx1, -1.0, p)
    m2 = jnp.max(p2, axis=1, keepdims=True)
    idx2 = jnp.min(jnp.where(p2 >= m2, lane, E), axis=1, keepdims=True)
    oh2 = (lane == idx2).astype(jnp.float32)
    tot = m1 + m2
    w1 = m1 / tot
    w2 = m2 / tot

    oh = oh1 + oh2
    ri = jax.lax.broadcasted_iota(jnp.int32, (BT, BT), 0)
    ci = jax.lax.broadcasted_iota(jnp.int32, (BT, BT), 1)
    tri = (ci < ri).astype(jnp.float32)
    cb = jnp.dot(tri, oh, preferred_element_type=jnp.float32) + cnt_ref[...]
    rank1 = jnp.sum(cb * oh1, axis=1, keepdims=True)
    rank2 = jnp.sum(cb * oh2, axis=1, keepdims=True)
    cnt_ref[...] += jnp.sum(oh, axis=0, keepdims=True)

    zi = jnp.zeros((BT, 4), jnp.int32)
    mi_out[...] = jnp.concatenate(
        [idx1, idx2, rank1.astype(jnp.int32), rank2.astype(jnp.int32), zi], axis=1)
    zf = jnp.zeros((BT, 6), jnp.float32)
    mf_out[...] = jnp.concatenate([w1, w2, zf], axis=1)

    @pl.when(i == NQ - 1)
    def _():
        cnt_out[...] = cnt_ref[...].astype(jnp.int32)


def _route_body(mi_ref, cnt_ref, pos_out, te_out):
    c = cnt_ref[...]
    nt = lax.div(c + (TILE - 1), TILE)
    e0 = jax.lax.broadcasted_iota(jnp.int32, (E, E), 0)
    e1 = jax.lax.broadcasted_iota(jnp.int32, (E, E), 1)
    up = (e0 < e1).astype(jnp.float32)
    base_tile = jnp.dot(nt.astype(jnp.float32), up,
                        preferred_element_type=jnp.float32).astype(jnp.int32)
    base_elem = base_tile * TILE

    mi = mi_ref[...]
    idx1 = mi[:, 0:1]
    idx2 = mi[:, 1:2]
    rank1 = mi[:, 2:3]
    rank2 = mi[:, 3:4]
    lane = jax.lax.broadcasted_iota(jnp.int32, (S, E), 1)
    be = jnp.broadcast_to(base_elem, (S, E))
    pos1 = jnp.sum(jnp.where(lane == idx1, be, 0), axis=1, keepdims=True) + rank1
    pos2 = jnp.sum(jnp.where(lane == idx2, be, 0), axis=1, keepdims=True) + rank2
    pos_out[...] = jnp.concatenate(
        [pos1, pos2, jnp.zeros((S, 6), jnp.int32)], axis=1)

    end_tile = base_tile + nt
    rowi = jax.lax.broadcasted_iota(jnp.int32, (NT, E), 0)
    ge = (rowi >= jnp.broadcast_to(end_tile, (NT, E))).astype(jnp.int32)
    te = jnp.minimum(jnp.sum(ge, axis=1, keepdims=True), E - 1)
    te_out[...] = jnp.broadcast_to(te, (NT, E))


def _sc_dispatch(h2, pos1, pos2):
    mesh = plsc.VectorSubcoreMesh(core_axis_name="c", subcore_axis_name="s")

    @functools.partial(
        pl.kernel, mesh=mesh,
        out_type=jax.ShapeDtypeStruct((NPAD, D), jnp.float32),
        scratch_types=[
            pltpu.VMEM((TPW, D), jnp.float32),
            pltpu.VMEM((TPW,), jnp.int32),
            pltpu.VMEM((TPW,), jnp.int32),
            pltpu.SemaphoreType.DMA,
        ],
    )
    def disp(h2_hbm, p1_hbm, p2_hbm, out_hbm, rows_v, p1_v, p2_v, sem):
        wid = lax.axis_index("s") * 2 + lax.axis_index("c")
        base = wid * TPW
        pltpu.sync_copy(h2_hbm.at[pl.ds(base, TPW)], rows_v)
        pltpu.sync_copy(p1_hbm.at[pl.ds(base, TPW)], p1_v)
        pltpu.sync_copy(p2_hbm.at[pl.ds(base, TPW)], p2_v)
        pltpu.async_copy(rows_v, out_hbm.at[p1_v], sem).wait()
        pltpu.async_copy(rows_v, out_hbm.at[p2_v], sem).wait()

    return disp(h2, pos1, pos2)


def _moe_body(te_ref, x_ref, wg_ref, wu_ref, wd_ref, y_ref,
              wgb_ref, wub_ref, wdb_ref):
    i = pl.program_id(0)
    changed = jnp.logical_or(
        i == 0, te_ref[i] != te_ref[jnp.maximum(i - 1, 0)])

    @pl.when(changed)
    def _():
        wgb_ref[...] = wg_ref[0].astype(jnp.bfloat16)
        wub_ref[...] = wu_ref[0].astype(jnp.bfloat16)
        wdb_ref[...] = wd_ref[0].astype(jnp.bfloat16)

    h2 = x_ref[...].astype(jnp.bfloat16)
    g = jnp.dot(h2, wgb_ref[...], preferred_element_type=jnp.float32)
    u = jnp.dot(h2, wub_ref[...], preferred_element_type=jnp.float32)
    hm = (g * jax.nn.sigmoid(g) * u).astype(jnp.bfloat16)
    y_ref[...] = jnp.dot(hm, wdb_ref[...], preferred_element_type=jnp.float32)


def _sc_combine(y, pos1, pos2):
    mesh = plsc.VectorSubcoreMesh(core_axis_name="c", subcore_axis_name="s")

    @functools.partial(
        pl.kernel, mesh=mesh,
        out_type=[
            jax.ShapeDtypeStruct((S, D), jnp.float32),
            jax.ShapeDtypeStruct((S, D), jnp.float32),
        ],
        scratch_types=[
            pltpu.VMEM((TPW, D), jnp.float32),
            pltpu.VMEM((TPW,), jnp.int32),
            pltpu.SemaphoreType.DMA,
        ],
    )
    def comb(y_hbm, p1_hbm, p2_hbm, y0_hbm, y1_hbm, rows_v, p_v, sem):
        wid = lax.axis_index("s") * 2 + lax.axis_index("c")
        base = wid * TPW
        pltpu.sync_copy(p1_hbm.at[pl.ds(base, TPW)], p_v)
        pltpu.async_copy(y_hbm.at[p_v], rows_v, sem).wait()
        pltpu.sync_copy(rows_v, y0_hbm.at[pl.ds(base, TPW)])
        pltpu.sync_copy(p2_hbm.at[pl.ds(base, TPW)], p_v)
        pltpu.async_copy(y_hbm.at[p_v], rows_v, sem).wait()
        pltpu.sync_copy(rows_v, y1_hbm.at[pl.ds(base, TPW)])

    return comb(y, pos1, pos2)


def _final_body(x_ref, y0_ref, y1_ref, mf_ref, out_ref):
    mf = mf_ref[...]
    out_ref[...] = (x_ref[...] + mf[:, 0:1] * y0_ref[...]
                    + mf[:, 1:2] * y1_ref[...])


def kernel(hidden_states, position_ids, ln1_w, ln2_w, Wq, bq, Wk, bk, Wv, bv,
           Wo, bo, router_w, Wg, Wu, Wd):
    x0 = hidden_states.reshape(S, D)

    r64 = np.zeros((HD, HD), np.float32)
    r64[np.arange(32) + 32, np.arange(32)] = -1.0
    r64[np.arange(32), np.arange(32) + 32] = 1.0
    Rq = jnp.asarray(np.kron(np.eye(H, dtype=np.float32), r64),
                     dtype=jnp.bfloat16)

    inv = jnp.asarray(1.0 / (THETA ** (np.arange(0, HD, 2, dtype=np.float32) / HD)))
    ang = position_ids.reshape(S, 1).astype(jnp.float32) * inv[None, :]
    cos64 = jnp.concatenate([jnp.cos(ang), jnp.cos(ang)], axis=1)
    sin64 = jnp.concatenate([jnp.sin(ang), jnp.sin(ang)], axis=1)
    cq = jnp.tile(cos64, (1, H))
    sq = jnp.tile(sin64, (1, H))

    rep = H // KV
    Wk_rep = jnp.repeat(Wk.reshape(D, KV, HD), rep, axis=1).reshape(D, H * HD)
    Wv_rep = jnp.repeat(Wv.reshape(D, KV, HD), rep, axis=1).reshape(D, H * HD)
    bk_rep = jnp.repeat(bk.reshape(KV, HD), rep, axis=0).reshape(H * HD)
    bv_rep = jnp.repeat(bv.reshape(KV, HD), rep, axis=0).reshape(H * HD)
    Wqkv = jnp.concatenate([Wq, Wk_rep, Wv_rep], axis=1).astype(jnp.bfloat16)
    bqkv = jnp.concatenate([bq, bk_rep, bv_rep]).reshape(1, 3 * H * HD)
    Wo_bf = Wo.astype(jnp.bfloat16)

    qh, kh, vh = pl.pallas_call(
        _preattn_body,
        grid=(NQ,),
        in_specs=[
            pl.BlockSpec((BT, D), lambda i: (i, 0)),
            pl.BlockSpec((D, 3 * H * HD), lambda i: (0, 0)),
            pl.BlockSpec((1, 3 * H * HD), lambda i: (0, 0)),
            pl.BlockSpec((H * HD, H * HD), lambda i: (0, 0)),
            pl.BlockSpec((BT, H * HD), lambda i: (i, 0)),
            pl.BlockSpec((BT, H * HD), lambda i: (i, 0)),
            pl.BlockSpec((1, D), lambda i: (0, 0)),
        ],
        out_specs=[
            pl.BlockSpec((BT, H * HD), lambda i: (i, 0)),
            pl.BlockSpec((BT, H * HD), lambda i: (i, 0)),
            pl.BlockSpec((BT, H * HD), lambda i: (i, 0)),
        ],
        out_shape=[
            jax.ShapeDtypeStruct((S, H * HD), jnp.bfloat16),
            jax.ShapeDtypeStruct((S, H * HD), jnp.bfloat16),
            jax.ShapeDtypeStruct((S, H * HD), jnp.bfloat16),
        ],
        interpret=_INTERP,
    )(x0, Wqkv, bqkv, Rq, cq, sq, ln1_w.reshape(1, D))

    ctx = pl.pallas_call(
        _attn_body,
        grid=(H // 2, NQ),
        in_specs=[
            pl.BlockSpec((BT, 2 * HD), lambda j, i: (i, j)),
            pl.BlockSpec((S, 2 * HD), lambda j, i: (0, j)),
            pl.BlockSpec((S, 2 * HD), lambda j, i: (0, j)),
        ],
        out_specs=pl.BlockSpec((BT, 2 * HD), lambda j, i: (i, j)),
        out_shape=jax.ShapeDtypeStruct((S, H * HD), jnp.bfloat16),
        interpret=_INTERP,
    )(qh, kh, vh)

    x, h2, mi, mf, cnt = pl.pallas_call(
        _postattn_body,
        grid=(NQ,),
        in_specs=[
            pl.BlockSpec((BT, H * HD), lambda i: (i, 0)),
            pl.BlockSpec((H * HD, D), lambda i: (0, 0)),
            pl.BlockSpec((1, D), lambda i: (0, 0)),
            pl.BlockSpec((BT, D), lambda i: (i, 0)),
            pl.BlockSpec((1, D), lambda i: (0, 0)),
            pl.BlockSpec((D, E), lambda i: (0, 0)),
        ],
        out_specs=[
            pl.BlockSpec((BT, D), lambda i: (i, 0)),
            pl.BlockSpec((BT, D), lambda i: (i, 0)),
            pl.BlockSpec((BT, E), lambda i: (i, 0)),
            pl.BlockSpec((BT, E), lambda i: (i, 0)),
            pl.BlockSpec((1, E), lambda i: (0, 0)),
        ],
        out_shape=[
            jax.ShapeDtypeStruct((S, D), jnp.float32),
            jax.ShapeDtypeStruct((S, D), jnp.float32),
            jax.ShapeDtypeStruct((S, E), jnp.int32),
            jax.ShapeDtypeStruct((S, E), jnp.float32),
            jax.ShapeDtypeStruct((1, E), jnp.int32),
        ],
        scratch_shapes=[pltpu.VMEM((1, E), jnp.float32)],
        interpret=_INTERP,
    )(ctx, Wo_bf, bo.reshape(1, D), x0, ln2_w.reshape(1, D), router_w)

    pos, te = pl.pallas_call(
        _route_body,
        grid=(1,),
        in_specs=[
            pl.BlockSpec((S, E), lambda i: (0, 0)),
            pl.BlockSpec((1, E), lambda i: (0, 0)),
        ],
        out_specs=[
            pl.BlockSpec((S, E), lambda i: (0, 0)),
            pl.BlockSpec((NT, E), lambda i: (0, 0)),
        ],
        out_shape=[
            jax.ShapeDtypeStruct((S, E), jnp.int32),
            jax.ShapeDtypeStruct((NT, E), jnp.int32),
        ],
        interpret=_INTERP,
    )(mi, cnt)

    pos1 = pos[:, 0]
    pos2 = pos[:, 1]
    te_arr = te[:, 0]

    x_disp = _sc_dispatch(h2, pos1, pos2)

    y = pl.pallas_call(
        _moe_body,
        grid_spec=pltpu.PrefetchScalarGridSpec(
            num_scalar_prefetch=1,
            grid=(NT,),
            in_specs=[
                pl.BlockSpec((TILE, D), lambda i, te_r: (i, 0)),
                pl.BlockSpec((1, D, DI), lambda i, te_r: (te_r[i], 0, 0)),
                pl.BlockSpec((1, D, DI), lambda i, te_r: (te_r[i], 0, 0)),
                pl.BlockSpec((1, DI, D), lambda i, te_r: (te_r[i], 0, 0)),
            ],
            out_specs=pl.BlockSpec((TILE, D), lambda i, te_r: (i, 0)),
            scratch_shapes=[
                pltpu.VMEM((D, DI), jnp.bfloat16),
                pltpu.VMEM((D, DI), jnp.bfloat16),
                pltpu.VMEM((DI, D), jnp.bfloat16),
            ],
        ),
        out_shape=jax.ShapeDtypeStruct((NPAD, D), jnp.float32),
        interpret=_INTERP,
    )(te_arr, x_disp, Wg, Wu, Wd)

    y0, y1 = _sc_combine(y, pos1, pos2)

    out = pl.pallas_call(
        _final_body,
        grid=(NQ,),
        in_specs=[
            pl.BlockSpec((BT, D), lambda i: (i, 0)),
            pl.BlockSpec((BT, D), lambda i: (i, 0)),
            pl.BlockSpec((BT, D), lambda i: (i, 0)),
            pl.BlockSpec((BT, E), lambda i: (i, 0)),
        ],
        out_specs=pl.BlockSpec((BT, D), lambda i: (i, 0)),
        out_shape=jax.ShapeDtypeStruct((S, D), jnp.float32),
        interpret=_INTERP,
    )(x, y0, y1, mf)

    return out.reshape(B, S, D)


# speedup vs baseline: 1.1861x; 1.1861x over previous
import functools

import numpy as np
import jax
from jax import lax
import jax.numpy as jnp
from jax.experimental import pallas as pl
from jax.experimental.pallas import tpu as pltpu
from jax.experimental.pallas import tpu_sc as plsc

B, S, D = 1, 2048, 1024
H, KV, HD = 16, 8, 64
E, TOPK, DI = 8, 2, 1024
EPS = 1e-05
THETA = 10000.0

BT = 256
NQ = S // BT
TILE = 128
NPAD = TOPK * S + E * TILE
NT = NPAD // TILE
NW = 32
TPW = S // NW

_INTERP = False


def _preattn_body(x_ref, w_ref, b_ref, rq_ref, cq_ref, sq_ref,
                  ln1_ref, q_out, k_out, v_out):
    x = x_ref[...]
    var = jnp.mean(x * x, axis=1, keepdims=True)
    h = (x * jax.lax.rsqrt(var + EPS) * ln1_ref[...]).astype(jnp.bfloat16)
    qkv = jnp.dot(h, w_ref[...], preferred_element_type=jnp.float32) + b_ref[...]
    hh = H * HD
    q = qkv[:, :hh]
    k = qkv[:, hh:2 * hh]
    v = qkv[:, 2 * hh:]
    cq, sq, rq = cq_ref[...], sq_ref[...], rq_ref[...]
    qr = jnp.dot(q.astype(jnp.bfloat16), rq, preferred_element_type=jnp.float32)
    kr = jnp.dot(k.astype(jnp.bfloat16), rq, preferred_element_type=jnp.float32)
    q_out[...] = ((q * cq + qr * sq) * 0.125).astype(jnp.bfloat16)
    k_out[...] = (k * cq + kr * sq).astype(jnp.bfloat16)
    v_out[...] = v.astype(jnp.bfloat16)


BK = 512


def _attn_body(q_ref, k_ref, v_ref, o_ref):
    iq = pl.program_id(1)
    q = q_ref[...]
    q2 = jnp.concatenate([q[:, :HD], q[:, HD:]], axis=0)
    row = jax.lax.broadcasted_iota(jnp.int32, (2 * BT, BK), 0)
    tok = iq * BT + jax.lax.rem(row, BT)
    col0 = jax.lax.broadcasted_iota(jnp.int32, (2 * BT, BK), 1)

    nchunks = iq // (BK // BT) + 1

    def body(kb, carry):
        acc, l = carry
        k_c = k_ref[pl.ds(kb * BK, BK), :HD]
        v_c = v_ref[pl.ds(kb * BK, BK), :HD]
        s = jax.lax.dot_general(q2, k_c, (((1,), (1,)), ((), ())),
                                preferred_element_type=jnp.float32)
        s = jnp.where(col0 + kb * BK <= tok, s, -1e9)
        e = jnp.exp(s)
        l = l + jnp.sum(e, axis=1, keepdims=True)
        acc = acc + jnp.dot(e.astype(jnp.bfloat16), v_c,
                            preferred_element_type=jnp.float32)
        return acc, l

    acc, l = jax.lax.fori_loop(
        0, nchunks, body,
        (jnp.zeros((2 * BT, HD), jnp.float32),
         jnp.zeros((2 * BT, 1), jnp.float32)))
    ctx2 = acc / l
    o_ref[...] = jnp.concatenate([ctx2[:BT], ctx2[BT:]], axis=1).astype(jnp.bfloat16)


def _postattn_body(ctx_ref, wo_ref, bo_ref, res_ref, ln2_ref, rw_ref,
                   x_out, h2_out, mi_out, mf_out, cnt_out, cnt_ref):
    i = pl.program_id(0)

    @pl.when(i == 0)
    def _():
        cnt_ref[...] = jnp.zeros((1, E), jnp.float32)

    xo = jnp.dot(ctx_ref[...], wo_ref[...], preferred_element_type=jnp.float32)
    x = res_ref[...] + xo + bo_ref[...]
    x_out[...] = x
    var = jnp.mean(x * x, axis=1, keepdims=True)
    h2 = x * jax.lax.rsqrt(var + EPS) * ln2_ref[...]
    h2_out[...] = h2
    logits = jnp.dot(h2, rw_ref[...], preferred_element_type=jnp.float32)
    lm = jnp.max(logits, axis=1, keepdims=True)
    ex = jnp.exp(logits - lm)
    p = ex / jnp.sum(ex, axis=1, keepdims=True)
    lane = jax.lax.broadcasted_iota(jnp.int32, (BT, E), 1)
    m1 = jnp.max(p, axis=1, keepdims=True)
    idx1 = jnp.min(jnp.where(p >= m1, lane, E), axis=1, keepdims=True)
    oh1 = (lane == idx1).astype(jnp.float32)
    p2 = jnp.where(lane == idx1, -1.0, p)
    m2 = jnp.max(p2, axis=1, keepdims=True)
    idx2 = jnp.min(jnp.where(p2 >= m2, lane, E), axis=1, keepdims=True)
    oh2 = (lane == idx2).astype(jnp.float32)
    tot = m1 + m2
    w1 = m1 / tot
    w2 = m2 / tot

    oh = oh1 + oh2
    ri = jax.lax.broadcasted_iota(jnp.int32, (BT, BT), 0)
    ci = jax.lax.broadcasted_iota(jnp.int32, (BT, BT), 1)
    tri = (ci < ri).astype(jnp.float32)
    cb = jnp.dot(tri, oh, preferred_element_type=jnp.float32) + cnt_ref[...]
    rank1 = jnp.sum(cb * oh1, axis=1, keepdims=True)
    rank2 = jnp.sum(cb * oh2, axis=1, keepdims=True)
    cnt_ref[...] += jnp.sum(oh, axis=0, keepdims=True)

    zi = jnp.zeros((BT, 4), jnp.int32)
    mi_out[...] = jnp.concatenate(
        [idx1, idx2, rank1.astype(jnp.int32), rank2.astype(jnp.int32), zi], axis=1)
    zf = jnp.zeros((BT, 6), jnp.float32)
    mf_out[...] = jnp.concatenate([w1, w2, zf], axis=1)

    @pl.when(i == NQ - 1)
    def _():
        cnt_out[...] = cnt_ref[...].astype(jnp.int32)


def _route_body(mi_ref, cnt_ref, pos_out, te_out):
    c = cnt_ref[...]
    nt = lax.div(c + (TILE - 1), TILE)
    e0 = jax.lax.broadcasted_iota(jnp.int32, (E, E), 0)
    e1 = jax.lax.broadcasted_iota(jnp.int32, (E, E), 1)
    up = (e0 < e1).astype(jnp.float32)
    base_tile = jnp.dot(nt.astype(jnp.float32), up,
                        preferred_element_type=jnp.float32).astype(jnp.int32)
    base_elem = base_tile * TILE

    mi = mi_ref[...]
    idx1 = mi[:, 0:1]
    idx2 = mi[:, 1:2]
    rank1 = mi[:, 2:3]
    rank2 = mi[:, 3:4]
    lane = jax.lax.broadcasted_iota(jnp.int32, (S, E), 1)
    be = jnp.broadcast_to(base_elem, (S, E))
    pos1 = jnp.sum(jnp.where(lane == idx1, be, 0), axis=1, keepdims=True) + rank1
    pos2 = jnp.sum(jnp.where(lane == idx2, be, 0), axis=1, keepdims=True) + rank2
    pos_out[...] = jnp.concatenate(
        [pos1, pos2, jnp.zeros((S, 6), jnp.int32)], axis=1)

    end_tile = base_tile + nt
    rowi = jax.lax.broadcasted_iota(jnp.int32, (NT, E), 0)
    ge = (rowi >= jnp.broadcast_to(end_tile, (NT, E))).astype(jnp.int32)
    te = jnp.minimum(jnp.sum(ge, axis=1, keepdims=True), E - 1)
    te_out[...] = jnp.broadcast_to(te, (NT, E))


def _sc_dispatch(h2, pos1, pos2):
    mesh = plsc.VectorSubcoreMesh(core_axis_name="c", subcore_axis_name="s")

    @functools.partial(
        pl.kernel, mesh=mesh,
        out_type=jax.ShapeDtypeStruct((NPAD, D), jnp.float32),
        scratch_types=[
            pltpu.VMEM((TPW, D), jnp.float32),
            pltpu.VMEM((TPW,), jnp.int32),
            pltpu.VMEM((TPW,), jnp.int32),
            pltpu.SemaphoreType.DMA,
        ],
    )
    def disp(h2_hbm, p1_hbm, p2_hbm, out_hbm, rows_v, p1_v, p2_v, sem):
        wid = lax.axis_index("s") * 2 + lax.axis_index("c")
        base = wid * TPW
        pltpu.sync_copy(h2_hbm.at[pl.ds(base, TPW)], rows_v)
        pltpu.sync_copy(p1_hbm.at[pl.ds(base, TPW)], p1_v)
        pltpu.sync_copy(p2_hbm.at[pl.ds(base, TPW)], p2_v)
        pltpu.async_copy(rows_v, out_hbm.at[p1_v], sem).wait()
        pltpu.async_copy(rows_v, out_hbm.at[p2_v], sem).wait()

    return disp(h2, pos1, pos2)


def _moe_body(te_ref, x_ref, wg_ref, wu_ref, wd_ref, y_ref,
              wgb_ref, wub_ref, wdb_ref):
    i = pl.program_id(0)
    changed = jnp.logical_or(
        i == 0, te_ref[i] != te_ref[jnp.maximum(i - 1, 0)])

    @pl.when(changed)
    def _():
        wgb_ref[...] = wg_ref[0].astype(jnp.bfloat16)
        wub_ref[...] = wu_ref[0].astype(jnp.bfloat16)
        wdb_ref[...] = wd_ref[0].astype(jnp.bfloat16)

    h2 = x_ref[...].astype(jnp.bfloat16)
    g = jnp.dot(h2, wgb_ref[...], preferred_element_type=jnp.float32)
    u = jnp.dot(h2, wub_ref[...], preferred_element_type=jnp.float32)
    hm = (g * jax.nn.sigmoid(g) * u).astype(jnp.bfloat16)
    y_ref[...] = jnp.dot(hm, wdb_ref[...], preferred_element_type=jnp.float32)


def _sc_combine(y, pos1, pos2):
    mesh = plsc.VectorSubcoreMesh(core_axis_name="c", subcore_axis_name="s")

    @functools.partial(
        pl.kernel, mesh=mesh,
        out_type=[
            jax.ShapeDtypeStruct((S, D), jnp.float32),
            jax.ShapeDtypeStruct((S, D), jnp.float32),
        ],
        scratch_types=[
            pltpu.VMEM((TPW, D), jnp.float32),
            pltpu.VMEM((TPW,), jnp.int32),
            pltpu.SemaphoreType.DMA,
        ],
    )
    def comb(y_hbm, p1_hbm, p2_hbm, y0_hbm, y1_hbm, rows_v, p_v, sem):
        wid = lax.axis_index("s") * 2 + lax.axis_index("c")
        base = wid * TPW
        pltpu.sync_copy(p1_hbm.at[pl.ds(base, TPW)], p_v)
        pltpu.async_copy(y_hbm.at[p_v], rows_v, sem).wait()
        pltpu.sync_copy(rows_v, y0_hbm.at[pl.ds(base, TPW)])
        pltpu.sync_copy(p2_hbm.at[pl.ds(base, TPW)], p_v)
        pltpu.async_copy(y_hbm.at[p_v], rows_v, sem).wait()
        pltpu.sync_copy(rows_v, y1_hbm.at[pl.ds(base, TPW)])

    return comb(y, pos1, pos2)


def _final_body(x_ref, y0_ref, y1_ref, mf_ref, out_ref):
    mf = mf_ref[...]
    out_ref[...] = (x_ref[...] + mf[:, 0:1] * y0_ref[...]
                    + mf[:, 1:2] * y1_ref[...])


def kernel(hidden_states, position_ids, ln1_w, ln2_w, Wq, bq, Wk, bk, Wv, bv,
           Wo, bo, router_w, Wg, Wu, Wd):
    x0 = hidden_states.reshape(S, D)

    r64 = np.zeros((HD, HD), np.float32)
    r64[np.arange(32) + 32, np.arange(32)] = -1.0
    r64[np.arange(32), np.arange(32) + 32] = 1.0
    Rq = jnp.asarray(np.kron(np.eye(H, dtype=np.float32), r64),
                     dtype=jnp.bfloat16)

    inv = jnp.asarray(1.0 / (THETA ** (np.arange(0, HD, 2, dtype=np.float32) / HD)))
    ang = position_ids.reshape(S, 1).astype(jnp.float32) * inv[None, :]
    cos64 = jnp.concatenate([jnp.cos(ang), jnp.cos(ang)], axis=1)
    sin64 = jnp.concatenate([jnp.sin(ang), jnp.sin(ang)], axis=1)
    cq = jnp.tile(cos64, (1, H))
    sq = jnp.tile(sin64, (1, H))

    rep = H // KV
    Wk_rep = jnp.repeat(Wk.reshape(D, KV, HD), rep, axis=1).reshape(D, H * HD)
    Wv_rep = jnp.repeat(Wv.reshape(D, KV, HD), rep, axis=1).reshape(D, H * HD)
    bk_rep = jnp.repeat(bk.reshape(KV, HD), rep, axis=0).reshape(H * HD)
    bv_rep = jnp.repeat(bv.reshape(KV, HD), rep, axis=0).reshape(H * HD)
    Wqkv = jnp.concatenate([Wq, Wk_rep, Wv_rep], axis=1).astype(jnp.bfloat16)
    bqkv = jnp.concatenate([bq, bk_rep, bv_rep]).reshape(1, 3 * H * HD)
    Wo_bf = Wo.astype(jnp.bfloat16)

    qh, kh, vh = pl.pallas_call(
        _preattn_body,
        grid=(NQ,),
        in_specs=[
            pl.BlockSpec((BT, D), lambda i: (i, 0)),
            pl.BlockSpec((D, 3 * H * HD), lambda i: (0, 0)),
            pl.BlockSpec((1, 3 * H * HD), lambda i: (0, 0)),
            pl.BlockSpec((H * HD, H * HD), lambda i: (0, 0)),
            pl.BlockSpec((BT, H * HD), lambda i: (i, 0)),
            pl.BlockSpec((BT, H * HD), lambda i: (i, 0)),
            pl.BlockSpec((1, D), lambda i: (0, 0)),
        ],
        out_specs=[
            pl.BlockSpec((BT, H * HD), lambda i: (i, 0)),
            pl.BlockSpec((BT, H * HD), lambda i: (i, 0)),
            pl.BlockSpec((BT, H * HD), lambda i: (i, 0)),
        ],
        out_shape=[
            jax.ShapeDtypeStruct((S, H * HD), jnp.bfloat16),
            jax.ShapeDtypeStruct((S, H * HD), jnp.bfloat16),
            jax.ShapeDtypeStruct((S, H * HD), jnp.bfloat16),
        ],
        interpret=_INTERP,
    )(x0, Wqkv, bqkv, Rq, cq, sq, ln1_w.reshape(1, D))

    ctx = pl.pallas_call(
        _attn_body,
        grid=(H // 2, NQ),
        in_specs=[
            pl.BlockSpec((BT, 2 * HD), lambda j, i: (i, j)),
            pl.BlockSpec((S, 2 * HD), lambda j, i: (0, j)),
            pl.BlockSpec((S, 2 * HD), lambda j, i: (0, j)),
        ],
        out_specs=pl.BlockSpec((BT, 2 * HD), lambda j, i: (i, j)),
        out_shape=jax.ShapeDtypeStruct((S, H * HD), jnp.bfloat16),
        interpret=_INTERP,
    )(qh, kh, vh)

    x, h2, mi, mf, cnt = pl.pallas_call(
        _postattn_body,
        grid=(NQ,),
        in_specs=[
            pl.BlockSpec((BT, H * HD), lambda i: (i, 0)),
            pl.BlockSpec((H * HD, D), lambda i: (0, 0)),
            pl.BlockSpec((1, D), lambda i: (0, 0)),
            pl.BlockSpec((BT, D), lambda i: (i, 0)),
            pl.BlockSpec((1, D), lambda i: (0, 0)),
            pl.BlockSpec((D, E), lambda i: (0, 0)),
        ],
        out_specs=[
            pl.BlockSpec((BT, D), lambda i: (i, 0)),
            pl.BlockSpec((BT, D), lambda i: (i, 0)),
            pl.BlockSpec((BT, E), lambda i: (i, 0)),
            pl.BlockSpec((BT, E), lambda i: (i, 0)),
            pl.BlockSpec((1, E), lambda i: (0, 0)),
        ],
        out_shape=[
            jax.ShapeDtypeStruct((S, D), jnp.float32),
            jax.ShapeDtypeStruct((S, D), jnp.float32),
            jax.ShapeDtypeStruct((S, E), jnp.int32),
            jax.ShapeDtypeStruct((S, E), jnp.float32),
            jax.ShapeDtypeStruct((1, E), jnp.int32),
        ],
        scratch_shapes=[pltpu.VMEM((1, E), jnp.float32)],
        interpret=_INTERP,
    )(ctx, Wo_bf, bo.reshape(1, D), x0, ln2_w.reshape(1, D), router_w)

    pos, te = pl.pallas_call(
        _route_body,
        grid=(1,),
        in_specs=[
            pl.BlockSpec((S, E), lambda i: (0, 0)),
            pl.BlockSpec((1, E), lambda i: (0, 0)),
        ],
        out_specs=[
            pl.BlockSpec((S, E), lambda i: (0, 0)),
            pl.BlockSpec((NT, E), lambda i: (0, 0)),
        ],
        out_shape=[
            jax.ShapeDtypeStruct((S, E), jnp.int32),
            jax.ShapeDtypeStruct((NT, E), jnp.int32),
        ],
        interpret=_INTERP,
    )(mi, cnt)

    pos1 = pos[:, 0]
    pos2 = pos[:, 1]
    te_arr = te[:, 0]

    x_disp = _sc_dispatch(h2, pos1, pos2)

    y = pl.pallas_call(
        _moe_body,
        grid_spec=pltpu.PrefetchScalarGridSpec(
            num_scalar_prefetch=1,
            grid=(NT,),
            in_specs=[
                pl.BlockSpec((TILE, D), lambda i, te_r: (i, 0)),
                pl.BlockSpec((1, D, DI), lambda i, te_r: (te_r[i], 0, 0)),
                pl.BlockSpec((1, D, DI), lambda i, te_r: (te_r[i], 0, 0)),
                pl.BlockSpec((1, DI, D), lambda i, te_r: (te_r[i], 0, 0)),
            ],
            out_specs=pl.BlockSpec((TILE, D), lambda i, te_r: (i, 0)),
            scratch_shapes=[
                pltpu.VMEM((D, DI), jnp.bfloat16),
                pltpu.VMEM((D, DI), jnp.bfloat16),
                pltpu.VMEM((DI, D), jnp.bfloat16),
            ],
        ),
        out_shape=jax.ShapeDtypeStruct((NPAD, D), jnp.float32),
        interpret=_INTERP,
    )(te_arr, x_disp, Wg, Wu, Wd)

    y0, y1 = _sc_combine(y, pos1, pos2)

    out = pl.pallas_call(
        _final_body,
        grid=(NQ,),
        in_specs=[
            pl.BlockSpec((BT, D), lambda i: (i, 0)),
            pl.BlockSpec((BT, D), lambda i: (i, 0)),
            pl.BlockSpec((BT, D), lambda i: (i, 0)),
            pl.BlockSpec((BT, E), lambda i: (i, 0)),
        ],
        out_specs=pl.BlockSpec((BT, D), lambda i: (i, 0)),
        out_shape=jax.ShapeDtypeStruct((S, D), jnp.float32),
        interpret=_INTERP,
    )(x, y0, y1, mf)

    return out.reshape(B, S, D)


# speedup vs baseline: 1.2294x; 1.0365x over previous
import functools

import numpy as np
import jax
from jax import lax
import jax.numpy as jnp
from jax.experimental import pallas as pl
from jax.experimental.pallas import tpu as pltpu
from jax.experimental.pallas import tpu_sc as plsc

B, S, D = 1, 2048, 1024
H, KV, HD = 16, 8, 64
E, TOPK, DI = 8, 2, 1024
EPS = 1e-05
THETA = 10000.0

BT = 256
NQ = S // BT
TILE = 256
NPAD = TOPK * S + E * TILE
NT = NPAD // TILE
NW = 32
TPW = S // NW

_INTERP = False


def _preattn_body(x_ref, w_ref, b_ref, rq_ref, cq_ref, sq_ref,
                  ln1_ref, q_out, k_out, v_out):
    x = x_ref[...]
    var = jnp.mean(x * x, axis=1, keepdims=True)
    h = (x * jax.lax.rsqrt(var + EPS) * ln1_ref[...]).astype(jnp.bfloat16)
    qkv = jnp.dot(h, w_ref[...], preferred_element_type=jnp.float32) + b_ref[...]
    hh = H * HD
    q = qkv[:, :hh]
    k = qkv[:, hh:2 * hh]
    v = qkv[:, 2 * hh:]
    cq, sq, rq = cq_ref[...], sq_ref[...], rq_ref[...]
    qr = jnp.dot(q.astype(jnp.bfloat16), rq, preferred_element_type=jnp.float32)
    kr = jnp.dot(k.astype(jnp.bfloat16), rq, preferred_element_type=jnp.float32)
    q_out[...] = ((q * cq + qr * sq) * 0.125).astype(jnp.bfloat16)
    k_out[...] = (k * cq + kr * sq).astype(jnp.bfloat16)
    v_out[...] = v.astype(jnp.bfloat16)


BK = 512


def _attn_body(q_ref, k_ref, v_ref, o_ref):
    iq = pl.program_id(1)
    q = q_ref[...]
    q2 = jnp.concatenate([q[:, :HD], q[:, HD:]], axis=0)
    row = jax.lax.broadcasted_iota(jnp.int32, (2 * BT, BK), 0)
    tok = iq * BT + jax.lax.rem(row, BT)
    col0 = jax.lax.broadcasted_iota(jnp.int32, (2 * BT, BK), 1)

    nfull = iq // (BK // BT)

    def body(kb, carry):
        acc, l = carry
        k_c = k_ref[pl.ds(kb * BK, BK), :HD]
        v_c = v_ref[pl.ds(kb * BK, BK), :HD]
        s = jax.lax.dot_general(q2, k_c, (((1,), (1,)), ((), ())),
                                preferred_element_type=jnp.float32)
        e = jnp.exp(s)
        l = l + jnp.sum(e, axis=1, keepdims=True)
        acc = acc + jnp.dot(e.astype(jnp.bfloat16), v_c,
                            preferred_element_type=jnp.float32)
        return acc, l

    acc, l = jax.lax.fori_loop(
        0, nfull, body,
        (jnp.zeros((2 * BT, HD), jnp.float32),
         jnp.zeros((2 * BT, 1), jnp.float32)))

    k_c = k_ref[pl.ds(nfull * BK, BK), :HD]
    v_c = v_ref[pl.ds(nfull * BK, BK), :HD]
    s = jax.lax.dot_general(q2, k_c, (((1,), (1,)), ((), ())),
                            preferred_element_type=jnp.float32)
    e = jnp.exp(jnp.where(col0 + nfull * BK <= tok, s, -1e9))
    l = l + jnp.sum(e, axis=1, keepdims=True)
    acc = acc + jnp.dot(e.astype(jnp.bfloat16), v_c,
                        preferred_element_type=jnp.float32)
    ctx2 = acc / l
    o_ref[...] = jnp.concatenate([ctx2[:BT], ctx2[BT:]], axis=1).astype(jnp.bfloat16)


def _postattn_body(ctx_ref, wo_ref, bo_ref, res_ref, ln2_ref, rw_ref,
                   x_out, h2_out, mi_out, mf_out, cnt_out, cnt_ref):
    i = pl.program_id(0)

    @pl.when(i == 0)
    def _():
        cnt_ref[...] = jnp.zeros((1, E), jnp.float32)

    xo = jnp.dot(ctx_ref[...], wo_ref[...], preferred_element_type=jnp.float32)
    x = res_ref[...] + xo + bo_ref[...]
    x_out[...] = x
    var = jnp.mean(x * x, axis=1, keepdims=True)
    h2 = x * jax.lax.rsqrt(var + EPS) * ln2_ref[...]
    h2_out[...] = h2
    logits = jnp.dot(h2, rw_ref[...], preferred_element_type=jnp.float32)
    lm = jnp.max(logits, axis=1, keepdims=True)
    ex = jnp.exp(logits - lm)
    p = ex / jnp.sum(ex, axis=1, keepdims=True)
    lane = jax.lax.broadcasted_iota(jnp.int32, (BT, E), 1)
    m1 = jnp.max(p, axis=1, keepdims=True)
    idx1 = jnp.min(jnp.where(p >= m1, lane, E), axis=1, keepdims=True)
    oh1 = (lane == idx1).astype(jnp.float32)
    p2 = jnp.where(lane == idx1, -1.0, p)
    m2 = jnp.max(p2, axis=1, keepdims=True)
    idx2 = jnp.min(jnp.where(p2 >= m2, lane, E), axis=1, keepdims=True)
    oh2 = (lane == idx2).astype(jnp.float32)
    tot = m1 + m2
    w1 = m1 / tot
    w2 = m2 / tot

    oh = oh1 + oh2
    ri = jax.lax.broadcasted_iota(jnp.int32, (BT, BT), 0)
    ci = jax.lax.broadcasted_iota(jnp.int32, (BT, BT), 1)
    tri = (ci < ri).astype(jnp.float32)
    cb = jnp.dot(tri, oh, preferred_element_type=jnp.float32) + cnt_ref[...]
    rank1 = jnp.sum(cb * oh1, axis=1, keepdims=True)
    rank2 = jnp.sum(cb * oh2, axis=1, keepdims=True)
    cnt_ref[...] += jnp.sum(oh, axis=0, keepdims=True)

    zi = jnp.zeros((BT, 4), jnp.int32)
    mi_out[...] = jnp.concatenate(
        [idx1, idx2, rank1.astype(jnp.int32), rank2.astype(jnp.int32), zi], axis=1)
    zf = jnp.zeros((BT, 6), jnp.float32)
    mf_out[...] = jnp.concatenate([w1, w2, zf], axis=1)

    @pl.when(i == NQ - 1)
    def _():
        cnt_out[...] = cnt_ref[...].astype(jnp.int32)


def _route_body(mi_ref, cnt_ref, pos_out, te_out):
    c = cnt_ref[...]
    nt = lax.div(c + (TILE - 1), TILE)
    e0 = jax.lax.broadcasted_iota(jnp.int32, (E, E), 0)
    e1 = jax.lax.broadcasted_iota(jnp.int32, (E, E), 1)
    up = (e0 < e1).astype(jnp.float32)
    base_tile = jnp.dot(nt.astype(jnp.float32), up,
                        preferred_element_type=jnp.float32).astype(jnp.int32)
    base_elem = base_tile * TILE

    mi = mi_ref[...]
    idx1 = mi[:, 0:1]
    idx2 = mi[:, 1:2]
    rank1 = mi[:, 2:3]
    rank2 = mi[:, 3:4]
    lane = jax.lax.broadcasted_iota(jnp.int32, (S, E), 1)
    be = jnp.broadcast_to(base_elem, (S, E))
    pos1 = jnp.sum(jnp.where(lane == idx1, be, 0), axis=1, keepdims=True) + rank1
    pos2 = jnp.sum(jnp.where(lane == idx2, be, 0), axis=1, keepdims=True) + rank2
    pos_out[...] = jnp.concatenate(
        [pos1, pos2, jnp.zeros((S, 6), jnp.int32)], axis=1)

    end_tile = base_tile + nt
    rowi = jax.lax.broadcasted_iota(jnp.int32, (NT, E), 0)
    ge = (rowi >= jnp.broadcast_to(end_tile, (NT, E))).astype(jnp.int32)
    te = jnp.minimum(jnp.sum(ge, axis=1, keepdims=True), E - 1)
    te_out[...] = jnp.broadcast_to(te, (NT, E))


def _sc_dispatch(h2, pos1, pos2):
    mesh = plsc.VectorSubcoreMesh(core_axis_name="c", subcore_axis_name="s")

    @functools.partial(
        pl.kernel, mesh=mesh,
        out_type=jax.ShapeDtypeStruct((NPAD, D), jnp.float32),
        scratch_types=[
            pltpu.VMEM((TPW, D), jnp.float32),
            pltpu.VMEM((TPW,), jnp.int32),
            pltpu.VMEM((TPW,), jnp.int32),
            pltpu.SemaphoreType.DMA,
        ],
    )
    def disp(h2_hbm, p1_hbm, p2_hbm, out_hbm, rows_v, p1_v, p2_v, sem):
        wid = lax.axis_index("s") * 2 + lax.axis_index("c")
        base = wid * TPW
        pltpu.sync_copy(h2_hbm.at[pl.ds(base, TPW)], rows_v)
        pltpu.sync_copy(p1_hbm.at[pl.ds(base, TPW)], p1_v)
        pltpu.sync_copy(p2_hbm.at[pl.ds(base, TPW)], p2_v)
        pltpu.async_copy(rows_v, out_hbm.at[p1_v], sem).wait()
        pltpu.async_copy(rows_v, out_hbm.at[p2_v], sem).wait()

    return disp(h2, pos1, pos2)


def _moe_body(te_ref, x_ref, wg_ref, wu_ref, wd_ref, y_ref,
              wgb_ref, wub_ref, wdb_ref):
    i = pl.program_id(0)
    changed = jnp.logical_or(
        i == 0, te_ref[i, 0] != te_ref[jnp.maximum(i - 1, 0), 0])

    @pl.when(changed)
    def _():
        wgb_ref[...] = wg_ref[0].astype(jnp.bfloat16)
        wub_ref[...] = wu_ref[0].astype(jnp.bfloat16)
        wdb_ref[...] = wd_ref[0].astype(jnp.bfloat16)

    h2 = x_ref[...].astype(jnp.bfloat16)
    g = jnp.dot(h2, wgb_ref[...], preferred_element_type=jnp.float32)
    u = jnp.dot(h2, wub_ref[...], preferred_element_type=jnp.float32)
    hm = (g * jax.nn.sigmoid(g) * u).astype(jnp.bfloat16)
    y_ref[...] = jnp.dot(hm, wdb_ref[...], preferred_element_type=jnp.float32)


def _sc_combine(y, pos1, pos2):
    mesh = plsc.VectorSubcoreMesh(core_axis_name="c", subcore_axis_name="s")

    @functools.partial(
        pl.kernel, mesh=mesh,
        out_type=[
            jax.ShapeDtypeStruct((S, D), jnp.float32),
            jax.ShapeDtypeStruct((S, D), jnp.float32),
        ],
        scratch_types=[
            pltpu.VMEM((TPW, D), jnp.float32),
            pltpu.VMEM((TPW,), jnp.int32),
            pltpu.SemaphoreType.DMA,
        ],
    )
    def comb(y_hbm, p1_hbm, p2_hbm, y0_hbm, y1_hbm, rows_v, p_v, sem):
        wid = lax.axis_index("s") * 2 + lax.axis_index("c")
        base = wid * TPW
        pltpu.sync_copy(p1_hbm.at[pl.ds(base, TPW)], p_v)
        pltpu.async_copy(y_hbm.at[p_v], rows_v, sem).wait()
        pltpu.sync_copy(rows_v, y0_hbm.at[pl.ds(base, TPW)])
        pltpu.sync_copy(p2_hbm.at[pl.ds(base, TPW)], p_v)
        pltpu.async_copy(y_hbm.at[p_v], rows_v, sem).wait()
        pltpu.sync_copy(rows_v, y1_hbm.at[pl.ds(base, TPW)])

    return comb(y, pos1, pos2)


def _final_body(x_ref, y0_ref, y1_ref, mf_ref, out_ref):
    mf = mf_ref[...]
    out_ref[...] = (x_ref[...] + mf[:, 0:1] * y0_ref[...]
                    + mf[:, 1:2] * y1_ref[...])


def kernel(hidden_states, position_ids, ln1_w, ln2_w, Wq, bq, Wk, bk, Wv, bv,
           Wo, bo, router_w, Wg, Wu, Wd):
    x0 = hidden_states.reshape(S, D)

    r64 = np.zeros((HD, HD), np.float32)
    r64[np.arange(32) + 32, np.arange(32)] = -1.0
    r64[np.arange(32), np.arange(32) + 32] = 1.0
    Rq = jnp.asarray(np.kron(np.eye(H, dtype=np.float32), r64),
                     dtype=jnp.bfloat16)

    inv = jnp.asarray(1.0 / (THETA ** (np.arange(0, HD, 2, dtype=np.float32) / HD)))
    ang = position_ids.reshape(S, 1).astype(jnp.float32) * inv[None, :]
    cos64 = jnp.concatenate([jnp.cos(ang), jnp.cos(ang)], axis=1)
    sin64 = jnp.concatenate([jnp.sin(ang), jnp.sin(ang)], axis=1)
    cq = jnp.tile(cos64, (1, H))
    sq = jnp.tile(sin64, (1, H))

    rep = H // KV
    Wk_rep = jnp.repeat(Wk.reshape(D, KV, HD), rep, axis=1).reshape(D, H * HD)
    Wv_rep = jnp.repeat(Wv.reshape(D, KV, HD), rep, axis=1).reshape(D, H * HD)
    bk_rep = jnp.repeat(bk.reshape(KV, HD), rep, axis=0).reshape(H * HD)
    bv_rep = jnp.repeat(bv.reshape(KV, HD), rep, axis=0).reshape(H * HD)
    Wqkv = jnp.concatenate([Wq, Wk_rep, Wv_rep], axis=1).astype(jnp.bfloat16)
    bqkv = jnp.concatenate([bq, bk_rep, bv_rep]).reshape(1, 3 * H * HD)
    Wo_bf = Wo.astype(jnp.bfloat16)

    qh, kh, vh = pl.pallas_call(
        _preattn_body,
        grid=(NQ,),
        in_specs=[
            pl.BlockSpec((BT, D), lambda i: (i, 0)),
            pl.BlockSpec((D, 3 * H * HD), lambda i: (0, 0)),
            pl.BlockSpec((1, 3 * H * HD), lambda i: (0, 0)),
            pl.BlockSpec((H * HD, H * HD), lambda i: (0, 0)),
            pl.BlockSpec((BT, H * HD), lambda i: (i, 0)),
            pl.BlockSpec((BT, H * HD), lambda i: (i, 0)),
            pl.BlockSpec((1, D), lambda i: (0, 0)),
        ],
        out_specs=[
            pl.BlockSpec((BT, H * HD), lambda i: (i, 0)),
            pl.BlockSpec((BT, H * HD), lambda i: (i, 0)),
            pl.BlockSpec((BT, H * HD), lambda i: (i, 0)),
        ],
        out_shape=[
            jax.ShapeDtypeStruct((S, H * HD), jnp.bfloat16),
            jax.ShapeDtypeStruct((S, H * HD), jnp.bfloat16),
            jax.ShapeDtypeStruct((S, H * HD), jnp.bfloat16),
        ],
        interpret=_INTERP,
    )(x0, Wqkv, bqkv, Rq, cq, sq, ln1_w.reshape(1, D))

    ctx = pl.pallas_call(
        _attn_body,
        grid=(H // 2, NQ),
        in_specs=[
            pl.BlockSpec((BT, 2 * HD), lambda j, i: (i, j)),
            pl.BlockSpec((S, 2 * HD), lambda j, i: (0, j)),
            pl.BlockSpec((S, 2 * HD), lambda j, i: (0, j)),
        ],
        out_specs=pl.BlockSpec((BT, 2 * HD), lambda j, i: (i, j)),
        out_shape=jax.ShapeDtypeStruct((S, H * HD), jnp.bfloat16),
        interpret=_INTERP,
    )(qh, kh, vh)

    x, h2, mi, mf, cnt = pl.pallas_call(
        _postattn_body,
        grid=(NQ,),
        in_specs=[
            pl.BlockSpec((BT, H * HD), lambda i: (i, 0)),
            pl.BlockSpec((H * HD, D), lambda i: (0, 0)),
            pl.BlockSpec((1, D), lambda i: (0, 0)),
            pl.BlockSpec((BT, D), lambda i: (i, 0)),
            pl.BlockSpec((1, D), lambda i: (0, 0)),
            pl.BlockSpec((D, E), lambda i: (0, 0)),
        ],
        out_specs=[
            pl.BlockSpec((BT, D), lambda i: (i, 0)),
            pl.BlockSpec((BT, D), lambda i: (i, 0)),
            pl.BlockSpec((BT, E), lambda i: (i, 0)),
            pl.BlockSpec((BT, E), lambda i: (i, 0)),
            pl.BlockSpec((1, E), lambda i: (0, 0)),
        ],
        out_shape=[
            jax.ShapeDtypeStruct((S, D), jnp.float32),
            jax.ShapeDtypeStruct((S, D), jnp.float32),
            jax.ShapeDtypeStruct((S, E), jnp.int32),
            jax.ShapeDtypeStruct((S, E), jnp.float32),
            jax.ShapeDtypeStruct((1, E), jnp.int32),
        ],
        scratch_shapes=[pltpu.VMEM((1, E), jnp.float32)],
        interpret=_INTERP,
    )(ctx, Wo_bf, bo.reshape(1, D), x0, ln2_w.reshape(1, D), router_w)

    pos, te = pl.pallas_call(
        _route_body,
        grid=(1,),
        in_specs=[
            pl.BlockSpec((S, E), lambda i: (0, 0)),
            pl.BlockSpec((1, E), lambda i: (0, 0)),
        ],
        out_specs=[
            pl.BlockSpec((S, E), lambda i: (0, 0)),
            pl.BlockSpec((NT, E), lambda i: (0, 0)),
        ],
        out_shape=[
            jax.ShapeDtypeStruct((S, E), jnp.int32),
            jax.ShapeDtypeStruct((NT, E), jnp.int32),
        ],
        interpret=_INTERP,
    )(mi, cnt)

    pos1 = pos[:, 0]
    pos2 = pos[:, 1]
    te_arr = te

    x_disp = _sc_dispatch(h2, pos1, pos2)

    y = pl.pallas_call(
        _moe_body,
        grid_spec=pltpu.PrefetchScalarGridSpec(
            num_scalar_prefetch=1,
            grid=(NT,),
            in_specs=[
                pl.BlockSpec((TILE, D), lambda i, te_r: (i, 0)),
                pl.BlockSpec((1, D, DI), lambda i, te_r: (te_r[i, 0], 0, 0)),
                pl.BlockSpec((1, D, DI), lambda i, te_r: (te_r[i, 0], 0, 0)),
                pl.BlockSpec((1, DI, D), lambda i, te_r: (te_r[i, 0], 0, 0)),
            ],
            out_specs=pl.BlockSpec((TILE, D), lambda i, te_r: (i, 0)),
            scratch_shapes=[
                pltpu.VMEM((D, DI), jnp.bfloat16),
                pltpu.VMEM((D, DI), jnp.bfloat16),
                pltpu.VMEM((DI, D), jnp.bfloat16),
            ],
        ),
        out_shape=jax.ShapeDtypeStruct((NPAD, D), jnp.float32),
        interpret=_INTERP,
    )(te_arr, x_disp, Wg, Wu, Wd)

    y0, y1 = _sc_combine(y, pos1, pos2)

    out = pl.pallas_call(
        _final_body,
        grid=(NQ,),
        in_specs=[
            pl.BlockSpec((BT, D), lambda i: (i, 0)),
            pl.BlockSpec((BT, D), lambda i: (i, 0)),
            pl.BlockSpec((BT, D), lambda i: (i, 0)),
            pl.BlockSpec((BT, E), lambda i: (i, 0)),
        ],
        out_specs=pl.BlockSpec((BT, D), lambda i: (i, 0)),
        out_shape=jax.ShapeDtypeStruct((S, D), jnp.float32),
        interpret=_INTERP,
    )(x, y0, y1, mf)

    return out.reshape(B, S, D)


# speedup vs baseline: 1.4069x; 1.1444x over previous
import functools

import numpy as np
import jax
from jax import lax
import jax.numpy as jnp
from jax.experimental import pallas as pl
from jax.experimental.pallas import tpu as pltpu
from jax.experimental.pallas import tpu_sc as plsc

B, S, D = 1, 2048, 1024
H, KV, HD = 16, 8, 64
E, TOPK, DI = 8, 2, 1024
EPS = 1e-05
THETA = 10000.0

BT = 256
NQ = S // BT
TILE = 256
NPAD = TOPK * S + E * TILE
NT = NPAD // TILE
NW = 32
TPW = S // NW

_INTERP = False


def _preattn_body(x_ref, wq_ref, wk_ref, wv_ref, b_ref, rq_ref, rk_ref,
                  c_ref, s_ref, ln1_ref, q_out, k_out, v_out,
                  wq_bf, wk_bf, wv_bf):
    i = pl.program_id(0)

    @pl.when(i == 0)
    def _():
        wq_bf[...] = wq_ref[...].astype(jnp.bfloat16)
        wk_bf[...] = wk_ref[...].astype(jnp.bfloat16)
        wv_bf[...] = wv_ref[...].astype(jnp.bfloat16)

    x = x_ref[...]
    var = jnp.mean(x * x, axis=1, keepdims=True)
    h = (x * jax.lax.rsqrt(var + EPS) * ln1_ref[...]).astype(jnp.bfloat16)
    b = b_ref[...]
    q = jnp.dot(h, wq_bf[...], preferred_element_type=jnp.float32) + b[:, :H * HD]
    k = (jnp.dot(h, wk_bf[...], preferred_element_type=jnp.float32)
         + b[:, H * HD:(H + KV) * HD])
    v = (jnp.dot(h, wv_bf[...], preferred_element_type=jnp.float32)
         + b[:, (H + KV) * HD:])
    c64, s64 = c_ref[...], s_ref[...]
    cq = jnp.concatenate([c64] * H, axis=1)
    sq = jnp.concatenate([s64] * H, axis=1)
    ck = jnp.concatenate([c64] * KV, axis=1)
    sk = jnp.concatenate([s64] * KV, axis=1)
    qr = jnp.dot(q.astype(jnp.bfloat16), rq_ref[...],
                 preferred_element_type=jnp.float32)
    kr = jnp.dot(k.astype(jnp.bfloat16), rk_ref[...],
                 preferred_element_type=jnp.float32)
    qf = ((q * cq + qr * sq) * 0.125).astype(jnp.bfloat16)
    kf = (k * ck + kr * sk).astype(jnp.bfloat16)
    vf = v.astype(jnp.bfloat16)
    q_out[...] = qf
    k_out[...] = jnp.concatenate(
        [kf[:, j * HD:(j + 1) * HD] for j in range(KV) for _ in range(2)], axis=1)
    v_out[...] = jnp.concatenate(
        [vf[:, j * HD:(j + 1) * HD] for j in range(KV) for _ in range(2)], axis=1)


BK = 512


def _attn_body(q_ref, k_ref, v_ref, o_ref):
    iq = pl.program_id(1)
    q = q_ref[...]
    q2 = jnp.concatenate([q[:, :HD], q[:, HD:]], axis=0)
    row = jax.lax.broadcasted_iota(jnp.int32, (2 * BT, BK), 0)
    tok = iq * BT + jax.lax.rem(row, BT)
    col0 = jax.lax.broadcasted_iota(jnp.int32, (2 * BT, BK), 1)

    nfull = iq // (BK // BT)

    def body(kb, carry):
        acc, l = carry
        k_c = k_ref[pl.ds(kb * BK, BK), :HD]
        v_c = v_ref[pl.ds(kb * BK, BK), :HD]
        s = jax.lax.dot_general(q2, k_c, (((1,), (1,)), ((), ())),
                                preferred_element_type=jnp.float32)
        e = jnp.exp(s)
        l = l + jnp.sum(e, axis=1, keepdims=True)
        acc = acc + jnp.dot(e.astype(jnp.bfloat16), v_c,
                            preferred_element_type=jnp.float32)
        return acc, l

    acc, l = jax.lax.fori_loop(
        0, nfull, body,
        (jnp.zeros((2 * BT, HD), jnp.float32),
         jnp.zeros((2 * BT, 1), jnp.float32)))

    k_c = k_ref[pl.ds(nfull * BK, BK), :HD]
    v_c = v_ref[pl.ds(nfull * BK, BK), :HD]
    s = jax.lax.dot_general(q2, k_c, (((1,), (1,)), ((), ())),
                            preferred_element_type=jnp.float32)
    e = jnp.exp(jnp.where(col0 + nfull * BK <= tok, s, -1e9))
    l = l + jnp.sum(e, axis=1, keepdims=True)
    acc = acc + jnp.dot(e.astype(jnp.bfloat16), v_c,
                        preferred_element_type=jnp.float32)
    ctx2 = acc / l
    o_ref[...] = jnp.concatenate([ctx2[:BT], ctx2[BT:]], axis=1).astype(jnp.bfloat16)


def _postattn_body(ctx_ref, wo_ref, bo_ref, res_ref, ln2_ref, rw_ref,
                   x_out, h2_out, mi_out, mf_out, cnt_out, cnt_ref):
    i = pl.program_id(0)

    @pl.when(i == 0)
    def _():
        cnt_ref[...] = jnp.zeros((1, E), jnp.float32)

    xo = jnp.dot(ctx_ref[...], wo_ref[...], preferred_element_type=jnp.float32)
    x = res_ref[...] + xo + bo_ref[...]
    x_out[...] = x
    var = jnp.mean(x * x, axis=1, keepdims=True)
    h2 = x * jax.lax.rsqrt(var + EPS) * ln2_ref[...]
    h2_out[...] = h2
    logits = jnp.dot(h2, rw_ref[...], preferred_element_type=jnp.float32)
    lm = jnp.max(logits, axis=1, keepdims=True)
    ex = jnp.exp(logits - lm)
    p = ex / jnp.sum(ex, axis=1, keepdims=True)
    lane = jax.lax.broadcasted_iota(jnp.int32, (BT, E), 1)
    m1 = jnp.max(p, axis=1, keepdims=True)
    idx1 = jnp.min(jnp.where(p >= m1, lane, E), axis=1, keepdims=True)
    oh1 = (lane == idx1).astype(jnp.float32)
    p2 = jnp.where(lane == idx1, -1.0, p)
    m2 = jnp.max(p2, axis=1, keepdims=True)
    idx2 = jnp.min(jnp.where(p2 >= m2, lane, E), axis=1, keepdims=True)
    oh2 = (lane == idx2).astype(jnp.float32)
    tot = m1 + m2
    w1 = m1 / tot
    w2 = m2 / tot

    oh = oh1 + oh2
    ri = jax.lax.broadcasted_iota(jnp.int32, (BT, BT), 0)
    ci = jax.lax.broadcasted_iota(jnp.int32, (BT, BT), 1)
    tri = (ci < ri).astype(jnp.float32)
    cb = jnp.dot(tri, oh, preferred_element_type=jnp.float32) + cnt_ref[...]
    rank1 = jnp.sum(cb * oh1, axis=1, keepdims=True)
    rank2 = jnp.sum(cb * oh2, axis=1, keepdims=True)
    cnt_ref[...] += jnp.sum(oh, axis=0, keepdims=True)

    zi = jnp.zeros((BT, 4), jnp.int32)
    mi_out[...] = jnp.concatenate(
        [idx1, idx2, rank1.astype(jnp.int32), rank2.astype(jnp.int32), zi], axis=1)
    zf = jnp.zeros((BT, 6), jnp.float32)
    mf_out[...] = jnp.concatenate([w1, w2, zf], axis=1)

    @pl.when(i == NQ - 1)
    def _():
        cnt_out[...] = cnt_ref[...].astype(jnp.int32)


def _route_body(mi_ref, cnt_ref, pos_out, te_out):
    c = cnt_ref[...]
    nt = lax.div(c + (TILE - 1), TILE)
    e0 = jax.lax.broadcasted_iota(jnp.int32, (E, E), 0)
    e1 = jax.lax.broadcasted_iota(jnp.int32, (E, E), 1)
    up = (e0 < e1).astype(jnp.float32)
    base_tile = jnp.dot(nt.astype(jnp.float32), up,
                        preferred_element_type=jnp.float32).astype(jnp.int32)
    base_elem = base_tile * TILE

    mi = mi_ref[...]
    idx1 = mi[:, 0:1]
    idx2 = mi[:, 1:2]
    rank1 = mi[:, 2:3]
    rank2 = mi[:, 3:4]
    lane = jax.lax.broadcasted_iota(jnp.int32, (S, E), 1)
    be = jnp.broadcast_to(base_elem, (S, E))
    pos1 = jnp.sum(jnp.where(lane == idx1, be, 0), axis=1, keepdims=True) + rank1
    pos2 = jnp.sum(jnp.where(lane == idx2, be, 0), axis=1, keepdims=True) + rank2
    pos_out[...] = jnp.concatenate(
        [pos1, pos2, jnp.zeros((S, 6), jnp.int32)], axis=1)

    end_tile = base_tile + nt
    rowi = jax.lax.broadcasted_iota(jnp.int32, (NT, E), 0)
    ge = (rowi >= jnp.broadcast_to(end_tile, (NT, E))).astype(jnp.int32)
    te = jnp.minimum(jnp.sum(ge, axis=1, keepdims=True), E - 1)
    te_out[...] = jnp.broadcast_to(te, (NT, E))


def _sc_dispatch(h2, pos1, pos2):
    mesh = plsc.VectorSubcoreMesh(core_axis_name="c", subcore_axis_name="s")

    @functools.partial(
        pl.kernel, mesh=mesh,
        out_type=jax.ShapeDtypeStruct((NPAD, D), jnp.float32),
        scratch_types=[
            pltpu.VMEM((TPW, D), jnp.float32),
            pltpu.VMEM((TPW,), jnp.int32),
            pltpu.VMEM((TPW,), jnp.int32),
            pltpu.SemaphoreType.DMA,
        ],
    )
    def disp(h2_hbm, p1_hbm, p2_hbm, out_hbm, rows_v, p1_v, p2_v, sem):
        wid = lax.axis_index("s") * 2 + lax.axis_index("c")
        base = wid * TPW
        pltpu.sync_copy(h2_hbm.at[pl.ds(base, TPW)], rows_v)
        pltpu.sync_copy(p1_hbm.at[pl.ds(base, TPW)], p1_v)
        pltpu.sync_copy(p2_hbm.at[pl.ds(base, TPW)], p2_v)
        pltpu.async_copy(rows_v, out_hbm.at[p1_v], sem).wait()
        pltpu.async_copy(rows_v, out_hbm.at[p2_v], sem).wait()

    return disp(h2, pos1, pos2)


def _moe_body(te_ref, x_ref, wg_ref, wu_ref, wd_ref, y_ref,
              wgb_ref, wub_ref, wdb_ref):
    i = pl.program_id(0)
    changed = jnp.logical_or(
        i == 0, te_ref[i, 0] != te_ref[jnp.maximum(i - 1, 0), 0])

    @pl.when(changed)
    def _():
        wgb_ref[...] = wg_ref[0].astype(jnp.bfloat16)
        wub_ref[...] = wu_ref[0].astype(jnp.bfloat16)
        wdb_ref[...] = wd_ref[0].astype(jnp.bfloat16)

    h2 = x_ref[...].astype(jnp.bfloat16)
    g = jnp.dot(h2, wgb_ref[...], preferred_element_type=jnp.float32)
    u = jnp.dot(h2, wub_ref[...], preferred_element_type=jnp.float32)
    hm = (g * jax.nn.sigmoid(g) * u).astype(jnp.bfloat16)
    y_ref[...] = jnp.dot(hm, wdb_ref[...], preferred_element_type=jnp.float32)


def _sc_combine(y, pos1, pos2):
    mesh = plsc.VectorSubcoreMesh(core_axis_name="c", subcore_axis_name="s")

    @functools.partial(
        pl.kernel, mesh=mesh,
        out_type=[
            jax.ShapeDtypeStruct((S, D), jnp.float32),
            jax.ShapeDtypeStruct((S, D), jnp.float32),
        ],
        scratch_types=[
            pltpu.VMEM((TPW, D), jnp.float32),
            pltpu.VMEM((TPW,), jnp.int32),
            pltpu.SemaphoreType.DMA,
        ],
    )
    def comb(y_hbm, p1_hbm, p2_hbm, y0_hbm, y1_hbm, rows_v, p_v, sem):
        wid = lax.axis_index("s") * 2 + lax.axis_index("c")
        base = wid * TPW
        pltpu.sync_copy(p1_hbm.at[pl.ds(base, TPW)], p_v)
        pltpu.async_copy(y_hbm.at[p_v], rows_v, sem).wait()
        pltpu.sync_copy(rows_v, y0_hbm.at[pl.ds(base, TPW)])
        pltpu.sync_copy(p2_hbm.at[pl.ds(base, TPW)], p_v)
        pltpu.async_copy(y_hbm.at[p_v], rows_v, sem).wait()
        pltpu.sync_copy(rows_v, y1_hbm.at[pl.ds(base, TPW)])

    return comb(y, pos1, pos2)


def _final_body(x_ref, y0_ref, y1_ref, mf_ref, out_ref):
    mf = mf_ref[...]
    out_ref[...] = (x_ref[...] + mf[:, 0:1] * y0_ref[...]
                    + mf[:, 1:2] * y1_ref[...])


def kernel(hidden_states, position_ids, ln1_w, ln2_w, Wq, bq, Wk, bk, Wv, bv,
           Wo, bo, router_w, Wg, Wu, Wd):
    x0 = hidden_states.reshape(S, D)

    r64 = np.zeros((HD, HD), np.float32)
    r64[np.arange(32) + 32, np.arange(32)] = -1.0
    r64[np.arange(32), np.arange(32) + 32] = 1.0
    Rq = jnp.asarray(np.kron(np.eye(H, dtype=np.float32), r64),
                     dtype=jnp.bfloat16)
    Rk = jnp.asarray(np.kron(np.eye(KV, dtype=np.float32), r64),
                     dtype=jnp.bfloat16)

    inv = jnp.asarray(1.0 / (THETA ** (np.arange(0, HD, 2, dtype=np.float32) / HD)))
    ang = position_ids.reshape(S, 1).astype(jnp.float32) * inv[None, :]
    cos64 = jnp.concatenate([jnp.cos(ang), jnp.cos(ang)], axis=1)
    sin64 = jnp.concatenate([jnp.sin(ang), jnp.sin(ang)], axis=1)

    bqkv = jnp.concatenate([bq, bk, bv]).reshape(1, (H + 2 * KV) * HD)
    Wo_bf = Wo.astype(jnp.bfloat16)

    qh, kh, vh = pl.pallas_call(
        _preattn_body,
        grid=(NQ,),
        in_specs=[
            pl.BlockSpec((BT, D), lambda i: (i, 0)),
            pl.BlockSpec((D, H * HD), lambda i: (0, 0)),
            pl.BlockSpec((D, KV * HD), lambda i: (0, 0)),
            pl.BlockSpec((D, KV * HD), lambda i: (0, 0)),
            pl.BlockSpec((1, (H + 2 * KV) * HD), lambda i: (0, 0)),
            pl.BlockSpec((H * HD, H * HD), lambda i: (0, 0)),
            pl.BlockSpec((KV * HD, KV * HD), lambda i: (0, 0)),
            pl.BlockSpec((BT, HD), lambda i: (i, 0)),
            pl.BlockSpec((BT, HD), lambda i: (i, 0)),
            pl.BlockSpec((1, D), lambda i: (0, 0)),
        ],
        out_specs=[
            pl.BlockSpec((BT, H * HD), lambda i: (i, 0)),
            pl.BlockSpec((BT, H * HD), lambda i: (i, 0)),
            pl.BlockSpec((BT, H * HD), lambda i: (i, 0)),
        ],
        out_shape=[
            jax.ShapeDtypeStruct((S, H * HD), jnp.bfloat16),
            jax.ShapeDtypeStruct((S, H * HD), jnp.bfloat16),
            jax.ShapeDtypeStruct((S, H * HD), jnp.bfloat16),
        ],
        scratch_shapes=[
            pltpu.VMEM((D, H * HD), jnp.bfloat16),
            pltpu.VMEM((D, KV * HD), jnp.bfloat16),
            pltpu.VMEM((D, KV * HD), jnp.bfloat16),
        ],
        interpret=_INTERP,
    )(x0, Wq, Wk, Wv, bqkv, Rq, Rk, cos64, sin64, ln1_w.reshape(1, D))

    ctx = pl.pallas_call(
        _attn_body,
        grid=(H // 2, NQ),
        in_specs=[
            pl.BlockSpec((BT, 2 * HD), lambda j, i: (i, j)),
            pl.BlockSpec((S, 2 * HD), lambda j, i: (0, j)),
            pl.BlockSpec((S, 2 * HD), lambda j, i: (0, j)),
        ],
        out_specs=pl.BlockSpec((BT, 2 * HD), lambda j, i: (i, j)),
        out_shape=jax.ShapeDtypeStruct((S, H * HD), jnp.bfloat16),
        interpret=_INTERP,
    )(qh, kh, vh)

    x, h2, mi, mf, cnt = pl.pallas_call(
        _postattn_body,
        grid=(NQ,),
        in_specs=[
            pl.BlockSpec((BT, H * HD), lambda i: (i, 0)),
            pl.BlockSpec((H * HD, D), lambda i: (0, 0)),
            pl.BlockSpec((1, D), lambda i: (0, 0)),
            pl.BlockSpec((BT, D), lambda i: (i, 0)),
            pl.BlockSpec((1, D), lambda i: (0, 0)),
            pl.BlockSpec((D, E), lambda i: (0, 0)),
        ],
        out_specs=[
            pl.BlockSpec((BT, D), lambda i: (i, 0)),
            pl.BlockSpec((BT, D), lambda i: (i, 0)),
            pl.BlockSpec((BT, E), lambda i: (i, 0)),
            pl.BlockSpec((BT, E), lambda i: (i, 0)),
            pl.BlockSpec((1, E), lambda i: (0, 0)),
        ],
        out_shape=[
            jax.ShapeDtypeStruct((S, D), jnp.float32),
            jax.ShapeDtypeStruct((S, D), jnp.float32),
            jax.ShapeDtypeStruct((S, E), jnp.int32),
            jax.ShapeDtypeStruct((S, E), jnp.float32),
            jax.ShapeDtypeStruct((1, E), jnp.int32),
        ],
        scratch_shapes=[pltpu.VMEM((1, E), jnp.float32)],
        interpret=_INTERP,
    )(ctx, Wo_bf, bo.reshape(1, D), x0, ln2_w.reshape(1, D), router_w)

    pos, te = pl.pallas_call(
        _route_body,
        grid=(1,),
        in_specs=[
            pl.BlockSpec((S, E), lambda i: (0, 0)),
            pl.BlockSpec((1, E), lambda i: (0, 0)),
        ],
        out_specs=[
            pl.BlockSpec((S, E), lambda i: (0, 0)),
            pl.BlockSpec((NT, E), lambda i: (0, 0)),
        ],
        out_shape=[
            jax.ShapeDtypeStruct((S, E), jnp.int32),
            jax.ShapeDtypeStruct((NT, E), jnp.int32),
        ],
        interpret=_INTERP,
    )(mi, cnt)

    pos1 = pos[:, 0]
    pos2 = pos[:, 1]
    te_arr = te

    x_disp = _sc_dispatch(h2, pos1, pos2)

    y = pl.pallas_call(
        _moe_body,
        grid_spec=pltpu.PrefetchScalarGridSpec(
            num_scalar_prefetch=1,
            grid=(NT,),
            in_specs=[
                pl.BlockSpec((TILE, D), lambda i, te_r: (i, 0)),
                pl.BlockSpec((1, D, DI), lambda i, te_r: (te_r[i, 0], 0, 0)),
                pl.BlockSpec((1, D, DI), lambda i, te_r: (te_r[i, 0], 0, 0)),
                pl.BlockSpec((1, DI, D), lambda i, te_r: (te_r[i, 0], 0, 0)),
            ],
            out_specs=pl.BlockSpec((TILE, D), lambda i, te_r: (i, 0)),
            scratch_shapes=[
                pltpu.VMEM((D, DI), jnp.bfloat16),
                pltpu.VMEM((D, DI), jnp.bfloat16),
                pltpu.VMEM((DI, D), jnp.bfloat16),
            ],
        ),
        out_shape=jax.ShapeDtypeStruct((NPAD, D), jnp.float32),
        interpret=_INTERP,
    )(te_arr, x_disp, Wg, Wu, Wd)

    y0, y1 = _sc_combine(y, pos1, pos2)

    out = pl.pallas_call(
        _final_body,
        grid=(NQ,),
        in_specs=[
            pl.BlockSpec((BT, D), lambda i: (i, 0)),
            pl.BlockSpec((BT, D), lambda i: (i, 0)),
            pl.BlockSpec((BT, D), lambda i: (i, 0)),
            pl.BlockSpec((BT, E), lambda i: (i, 0)),
        ],
        out_specs=pl.BlockSpec((BT, D), lambda i: (i, 0)),
        out_shape=jax.ShapeDtypeStruct((S, D), jnp.float32),
        interpret=_INTERP,
    )(x, y0, y1, mf)

    return out.reshape(B, S, D)


# speedup vs baseline: 1.4128x; 1.0042x over previous
import functools

import numpy as np
import jax
from jax import lax
import jax.numpy as jnp
from jax.experimental import pallas as pl
from jax.experimental.pallas import tpu as pltpu
from jax.experimental.pallas import tpu_sc as plsc

B, S, D = 1, 2048, 1024
H, KV, HD = 16, 8, 64
E, TOPK, DI = 8, 2, 1024
EPS = 1e-05
THETA = 10000.0

BT = 256
NQ = S // BT
TILE = 256
NPAD = TOPK * S + E * TILE
NT = NPAD // TILE
NW = 32
TPW = S // NW

_INTERP = False


def _preattn_body(x_ref, wq_ref, wk_ref, wv_ref, b_ref, rq_ref, rk_ref,
                  c_ref, s_ref, ln1_ref, q_out, k_out, v_out,
                  wq_bf, wk_bf, wv_bf):
    i = pl.program_id(0)

    @pl.when(i == 0)
    def _():
        wq_bf[...] = wq_ref[...].astype(jnp.bfloat16)
        wk_bf[...] = wk_ref[...].astype(jnp.bfloat16)
        wv_bf[...] = wv_ref[...].astype(jnp.bfloat16)

    x = x_ref[...]
    var = jnp.mean(x * x, axis=1, keepdims=True)
    h = (x * jax.lax.rsqrt(var + EPS) * ln1_ref[...]).astype(jnp.bfloat16)
    b = b_ref[...]
    q = jnp.dot(h, wq_bf[...], preferred_element_type=jnp.float32) + b[:, :H * HD]
    k = (jnp.dot(h, wk_bf[...], preferred_element_type=jnp.float32)
         + b[:, H * HD:(H + KV) * HD])
    v = (jnp.dot(h, wv_bf[...], preferred_element_type=jnp.float32)
         + b[:, (H + KV) * HD:])
    c64, s64 = c_ref[...], s_ref[...]
    cq = jnp.concatenate([c64] * H, axis=1)
    sq = jnp.concatenate([s64] * H, axis=1)
    ck = jnp.concatenate([c64] * KV, axis=1)
    sk = jnp.concatenate([s64] * KV, axis=1)
    qr = jnp.dot(q.astype(jnp.bfloat16), rq_ref[...],
                 preferred_element_type=jnp.float32)
    kr = jnp.dot(k.astype(jnp.bfloat16), rk_ref[...],
                 preferred_element_type=jnp.float32)
    qf = ((q * cq + qr * sq) * (0.125 * 1.4426950408889634)).astype(jnp.bfloat16)
    kf = (k * ck + kr * sk).astype(jnp.bfloat16)
    vf = v.astype(jnp.bfloat16)
    q_out[...] = qf
    k_out[...] = jnp.concatenate(
        [kf[:, j * HD:(j + 1) * HD] for j in range(KV) for _ in range(2)], axis=1)
    v_out[...] = jnp.concatenate(
        [vf[:, j * HD:(j + 1) * HD] for j in range(KV) for _ in range(2)], axis=1)


BK = 512


def _attn_body(q_ref, k_ref, v_ref, o_ref):
    iq = pl.program_id(1)
    q = q_ref[...]
    q2 = jnp.concatenate([q[:, :HD], q[:, HD:]], axis=0)
    row = jax.lax.broadcasted_iota(jnp.int32, (2 * BT, BK), 0)
    tok = iq * BT + jax.lax.rem(row, BT)
    col0 = jax.lax.broadcasted_iota(jnp.int32, (2 * BT, BK), 1)

    nfull = iq // (BK // BT)

    def body(kb, carry):
        acc, l = carry
        k_c = k_ref[pl.ds(kb * BK, BK), :HD]
        v_c = v_ref[pl.ds(kb * BK, BK), :HD]
        s = jax.lax.dot_general(q2, k_c, (((1,), (1,)), ((), ())),
                                preferred_element_type=jnp.float32)
        e = jnp.exp2(s)
        l = l + jnp.sum(e, axis=1, keepdims=True)
        acc = acc + jnp.dot(e.astype(jnp.bfloat16), v_c,
                            preferred_element_type=jnp.float32)
        return acc, l

    acc, l = jax.lax.fori_loop(
        0, nfull, body,
        (jnp.zeros((2 * BT, HD), jnp.float32),
         jnp.zeros((2 * BT, 1), jnp.float32)))

    k_c = k_ref[pl.ds(nfull * BK, BK), :HD]
    v_c = v_ref[pl.ds(nfull * BK, BK), :HD]
    s = jax.lax.dot_general(q2, k_c, (((1,), (1,)), ((), ())),
                            preferred_element_type=jnp.float32)
    e = jnp.exp2(jnp.where(col0 + nfull * BK <= tok, s, -1e9))
    l = l + jnp.sum(e, axis=1, keepdims=True)
    acc = acc + jnp.dot(e.astype(jnp.bfloat16), v_c,
                        preferred_element_type=jnp.float32)
    ctx2 = acc / l
    o_ref[...] = jnp.concatenate([ctx2[:BT], ctx2[BT:]], axis=1).astype(jnp.bfloat16)


def _postattn_body(ctx_ref, wo_ref, bo_ref, res_ref, ln2_ref, rw_ref,
                   x_out, h2_out, mi_out, mf_out, cnt_out, cnt_ref):
    i = pl.program_id(0)

    @pl.when(i == 0)
    def _():
        cnt_ref[...] = jnp.zeros((1, E), jnp.float32)

    xo = jnp.dot(ctx_ref[...], wo_ref[...], preferred_element_type=jnp.float32)
    x = res_ref[...] + xo + bo_ref[...]
    x_out[...] = x
    var = jnp.mean(x * x, axis=1, keepdims=True)
    h2 = x * jax.lax.rsqrt(var + EPS) * ln2_ref[...]
    h2_out[...] = h2
    logits = jnp.dot(h2, rw_ref[...], preferred_element_type=jnp.float32)
    lm = jnp.max(logits, axis=1, keepdims=True)
    ex = jnp.exp(logits - lm)
    p = ex / jnp.sum(ex, axis=1, keepdims=True)
    lane = jax.lax.broadcasted_iota(jnp.int32, (BT, E), 1)
    m1 = jnp.max(p, axis=1, keepdims=True)
    idx1 = jnp.min(jnp.where(p >= m1, lane, E), axis=1, keepdims=True)
    oh1 = (lane == idx1).astype(jnp.float32)
    p2 = jnp.where(lane == idx1, -1.0, p)
    m2 = jnp.max(p2, axis=1, keepdims=True)
    idx2 = jnp.min(jnp.where(p2 >= m2, lane, E), axis=1, keepdims=True)
    oh2 = (lane == idx2).astype(jnp.float32)
    tot = m1 + m2
    w1 = m1 / tot
    w2 = m2 / tot

    oh = oh1 + oh2
    ri = jax.lax.broadcasted_iota(jnp.int32, (BT, BT), 0)
    ci = jax.lax.broadcasted_iota(jnp.int32, (BT, BT), 1)
    tri = (ci < ri).astype(jnp.float32)
    cb = jnp.dot(tri, oh, preferred_element_type=jnp.float32) + cnt_ref[...]
    rank1 = jnp.sum(cb * oh1, axis=1, keepdims=True)
    rank2 = jnp.sum(cb * oh2, axis=1, keepdims=True)
    cnt_ref[...] += jnp.sum(oh, axis=0, keepdims=True)

    zi = jnp.zeros((BT, 4), jnp.int32)
    mi_out[...] = jnp.concatenate(
        [idx1, idx2, rank1.astype(jnp.int32), rank2.astype(jnp.int32), zi], axis=1)
    zf = jnp.zeros((BT, 6), jnp.float32)
    mf_out[...] = jnp.concatenate([w1, w2, zf], axis=1)

    @pl.when(i == NQ - 1)
    def _():
        cnt_out[...] = cnt_ref[...].astype(jnp.int32)


def _route_body(mi_ref, cnt_ref, pos_out, te_out):
    c = cnt_ref[...]
    nt = lax.div(c + (TILE - 1), TILE)
    e0 = jax.lax.broadcasted_iota(jnp.int32, (E, E), 0)
    e1 = jax.lax.broadcasted_iota(jnp.int32, (E, E), 1)
    up = (e0 < e1).astype(jnp.float32)
    base_tile = jnp.dot(nt.astype(jnp.float32), up,
                        preferred_element_type=jnp.float32).astype(jnp.int32)
    base_elem = base_tile * TILE

    mi = mi_ref[...]
    idx1 = mi[:, 0:1]
    idx2 = mi[:, 1:2]
    rank1 = mi[:, 2:3]
    rank2 = mi[:, 3:4]
    lane = jax.lax.broadcasted_iota(jnp.int32, (S, E), 1)
    be = jnp.broadcast_to(base_elem, (S, E))
    pos1 = jnp.sum(jnp.where(lane == idx1, be, 0), axis=1, keepdims=True) + rank1
    pos2 = jnp.sum(jnp.where(lane == idx2, be, 0), axis=1, keepdims=True) + rank2
    pos_out[...] = jnp.concatenate(
        [pos1, pos2, jnp.zeros((S, 6), jnp.int32)], axis=1)

    end_tile = base_tile + nt
    rowi = jax.lax.broadcasted_iota(jnp.int32, (NT, E), 0)
    ge = (rowi >= jnp.broadcast_to(end_tile, (NT, E))).astype(jnp.int32)
    te = jnp.minimum(jnp.sum(ge, axis=1, keepdims=True), E - 1)
    te_out[...] = jnp.broadcast_to(te, (NT, E))


def _sc_dispatch(h2, pos1, pos2):
    mesh = plsc.VectorSubcoreMesh(core_axis_name="c", subcore_axis_name="s")

    @functools.partial(
        pl.kernel, mesh=mesh,
        out_type=jax.ShapeDtypeStruct((NPAD, D), jnp.float32),
        scratch_types=[
            pltpu.VMEM((TPW, D), jnp.float32),
            pltpu.VMEM((TPW,), jnp.int32),
            pltpu.VMEM((TPW,), jnp.int32),
            pltpu.SemaphoreType.DMA,
        ],
    )
    def disp(h2_hbm, p1_hbm, p2_hbm, out_hbm, rows_v, p1_v, p2_v, sem):
        wid = lax.axis_index("s") * 2 + lax.axis_index("c")
        base = wid * TPW
        pltpu.sync_copy(h2_hbm.at[pl.ds(base, TPW)], rows_v)
        pltpu.sync_copy(p1_hbm.at[pl.ds(base, TPW)], p1_v)
        pltpu.sync_copy(p2_hbm.at[pl.ds(base, TPW)], p2_v)
        pltpu.async_copy(rows_v, out_hbm.at[p1_v], sem).wait()
        pltpu.async_copy(rows_v, out_hbm.at[p2_v], sem).wait()

    return disp(h2, pos1, pos2)


def _moe_body(te_ref, x_ref, wg_ref, wu_ref, wd_ref, y_ref,
              wgb_ref, wub_ref, wdb_ref):
    i = pl.program_id(0)
    changed = jnp.logical_or(
        i == 0, te_ref[i, 0] != te_ref[jnp.maximum(i - 1, 0), 0])

    @pl.when(changed)
    def _():
        wgb_ref[...] = wg_ref[0].astype(jnp.bfloat16)
        wub_ref[...] = wu_ref[0].astype(jnp.bfloat16)
        wdb_ref[...] = wd_ref[0].astype(jnp.bfloat16)

    h2 = x_ref[...].astype(jnp.bfloat16)
    g = jnp.dot(h2, wgb_ref[...], preferred_element_type=jnp.float32)
    u = jnp.dot(h2, wub_ref[...], preferred_element_type=jnp.float32)
    hm = (g * jax.nn.sigmoid(g) * u).astype(jnp.bfloat16)
    y_ref[...] = jnp.dot(hm, wdb_ref[...], preferred_element_type=jnp.float32)


def _sc_combine(y, pos1, pos2):
    mesh = plsc.VectorSubcoreMesh(core_axis_name="c", subcore_axis_name="s")

    @functools.partial(
        pl.kernel, mesh=mesh,
        out_type=[
            jax.ShapeDtypeStruct((S, D), jnp.float32),
            jax.ShapeDtypeStruct((S, D), jnp.float32),
        ],
        scratch_types=[
            pltpu.VMEM((TPW, D), jnp.float32),
            pltpu.VMEM((TPW,), jnp.int32),
            pltpu.SemaphoreType.DMA,
        ],
    )
    def comb(y_hbm, p1_hbm, p2_hbm, y0_hbm, y1_hbm, rows_v, p_v, sem):
        wid = lax.axis_index("s") * 2 + lax.axis_index("c")
        base = wid * TPW
        pltpu.sync_copy(p1_hbm.at[pl.ds(base, TPW)], p_v)
        pltpu.async_copy(y_hbm.at[p_v], rows_v, sem).wait()
        pltpu.sync_copy(rows_v, y0_hbm.at[pl.ds(base, TPW)])
        pltpu.sync_copy(p2_hbm.at[pl.ds(base, TPW)], p_v)
        pltpu.async_copy(y_hbm.at[p_v], rows_v, sem).wait()
        pltpu.sync_copy(rows_v, y1_hbm.at[pl.ds(base, TPW)])

    return comb(y, pos1, pos2)


def _final_body(x_ref, y0_ref, y1_ref, mf_ref, out_ref):
    mf = mf_ref[...]
    out_ref[...] = (x_ref[...] + mf[:, 0:1] * y0_ref[...]
                    + mf[:, 1:2] * y1_ref[...])


def kernel(hidden_states, position_ids, ln1_w, ln2_w, Wq, bq, Wk, bk, Wv, bv,
           Wo, bo, router_w, Wg, Wu, Wd):
    x0 = hidden_states.reshape(S, D)

    r64 = np.zeros((HD, HD), np.float32)
    r64[np.arange(32) + 32, np.arange(32)] = -1.0
    r64[np.arange(32), np.arange(32) + 32] = 1.0
    Rq = jnp.asarray(np.kron(np.eye(H, dtype=np.float32), r64),
                     dtype=jnp.bfloat16)
    Rk = jnp.asarray(np.kron(np.eye(KV, dtype=np.float32), r64),
                     dtype=jnp.bfloat16)

    inv = jnp.asarray(1.0 / (THETA ** (np.arange(0, HD, 2, dtype=np.float32) / HD)))
    ang = position_ids.reshape(S, 1).astype(jnp.float32) * inv[None, :]
    cos64 = jnp.concatenate([jnp.cos(ang), jnp.cos(ang)], axis=1)
    sin64 = jnp.concatenate([jnp.sin(ang), jnp.sin(ang)], axis=1)

    bqkv = jnp.concatenate([bq, bk, bv]).reshape(1, (H + 2 * KV) * HD)
    Wo_bf = Wo.astype(jnp.bfloat16)

    qh, kh, vh = pl.pallas_call(
        _preattn_body,
        grid=(NQ,),
        in_specs=[
            pl.BlockSpec((BT, D), lambda i: (i, 0)),
            pl.BlockSpec((D, H * HD), lambda i: (0, 0)),
            pl.BlockSpec((D, KV * HD), lambda i: (0, 0)),
            pl.BlockSpec((D, KV * HD), lambda i: (0, 0)),
            pl.BlockSpec((1, (H + 2 * KV) * HD), lambda i: (0, 0)),
            pl.BlockSpec((H * HD, H * HD), lambda i: (0, 0)),
            pl.BlockSpec((KV * HD, KV * HD), lambda i: (0, 0)),
            pl.BlockSpec((BT, HD), lambda i: (i, 0)),
            pl.BlockSpec((BT, HD), lambda i: (i, 0)),
            pl.BlockSpec((1, D), lambda i: (0, 0)),
        ],
        out_specs=[
            pl.BlockSpec((BT, H * HD), lambda i: (i, 0)),
            pl.BlockSpec((BT, H * HD), lambda i: (i, 0)),
            pl.BlockSpec((BT, H * HD), lambda i: (i, 0)),
        ],
        out_shape=[
            jax.ShapeDtypeStruct((S, H * HD), jnp.bfloat16),
            jax.ShapeDtypeStruct((S, H * HD), jnp.bfloat16),
            jax.ShapeDtypeStruct((S, H * HD), jnp.bfloat16),
        ],
        scratch_shapes=[
            pltpu.VMEM((D, H * HD), jnp.bfloat16),
            pltpu.VMEM((D, KV * HD), jnp.bfloat16),
            pltpu.VMEM((D, KV * HD), jnp.bfloat16),
        ],
        interpret=_INTERP,
    )(x0, Wq, Wk, Wv, bqkv, Rq, Rk, cos64, sin64, ln1_w.reshape(1, D))

    ctx = pl.pallas_call(
        _attn_body,
        grid=(H // 2, NQ),
        in_specs=[
            pl.BlockSpec((BT, 2 * HD), lambda j, i: (i, j)),
            pl.BlockSpec((S, 2 * HD), lambda j, i: (0, j)),
            pl.BlockSpec((S, 2 * HD), lambda j, i: (0, j)),
        ],
        out_specs=pl.BlockSpec((BT, 2 * HD), lambda j, i: (i, j)),
        out_shape=jax.ShapeDtypeStruct((S, H * HD), jnp.bfloat16),
        interpret=_INTERP,
    )(qh, kh, vh)

    x, h2, mi, mf, cnt = pl.pallas_call(
        _postattn_body,
        grid=(NQ,),
        in_specs=[
            pl.BlockSpec((BT, H * HD), lambda i: (i, 0)),
            pl.BlockSpec((H * HD, D), lambda i: (0, 0)),
            pl.BlockSpec((1, D), lambda i: (0, 0)),
            pl.BlockSpec((BT, D), lambda i: (i, 0)),
            pl.BlockSpec((1, D), lambda i: (0, 0)),
            pl.BlockSpec((D, E), lambda i: (0, 0)),
        ],
        out_specs=[
            pl.BlockSpec((BT, D), lambda i: (i, 0)),
            pl.BlockSpec((BT, D), lambda i: (i, 0)),
            pl.BlockSpec((BT, E), lambda i: (i, 0)),
            pl.BlockSpec((BT, E), lambda i: (i, 0)),
            pl.BlockSpec((1, E), lambda i: (0, 0)),
        ],
        out_shape=[
            jax.ShapeDtypeStruct((S, D), jnp.float32),
            jax.ShapeDtypeStruct((S, D), jnp.float32),
            jax.ShapeDtypeStruct((S, E), jnp.int32),
            jax.ShapeDtypeStruct((S, E), jnp.float32),
            jax.ShapeDtypeStruct((1, E), jnp.int32),
        ],
        scratch_shapes=[pltpu.VMEM((1, E), jnp.float32)],
        interpret=_INTERP,
    )(ctx, Wo_bf, bo.reshape(1, D), x0, ln2_w.reshape(1, D), router_w)

    pos, te = pl.pallas_call(
        _route_body,
        grid=(1,),
        in_specs=[
            pl.BlockSpec((S, E), lambda i: (0, 0)),
            pl.BlockSpec((1, E), lambda i: (0, 0)),
        ],
        out_specs=[
            pl.BlockSpec((S, E), lambda i: (0, 0)),
            pl.BlockSpec((NT, E), lambda i: (0, 0)),
        ],
        out_shape=[
            jax.ShapeDtypeStruct((S, E), jnp.int32),
            jax.ShapeDtypeStruct((NT, E), jnp.int32),
        ],
        interpret=_INTERP,
    )(mi, cnt)

    pos1 = pos[:, 0]
    pos2 = pos[:, 1]
    te_arr = te

    x_disp = _sc_dispatch(h2, pos1, pos2)

    y = pl.pallas_call(
        _moe_body,
        grid_spec=pltpu.PrefetchScalarGridSpec(
            num_scalar_prefetch=1,
            grid=(NT,),
            in_specs=[
                pl.BlockSpec((TILE, D), lambda i, te_r: (i, 0)),
                pl.BlockSpec((1, D, DI), lambda i, te_r: (te_r[i, 0], 0, 0)),
                pl.BlockSpec((1, D, DI), lambda i, te_r: (te_r[i, 0], 0, 0)),
                pl.BlockSpec((1, DI, D), lambda i, te_r: (te_r[i, 0], 0, 0)),
            ],
            out_specs=pl.BlockSpec((TILE, D), lambda i, te_r: (i, 0)),
            scratch_shapes=[
                pltpu.VMEM((D, DI), jnp.bfloat16),
                pltpu.VMEM((D, DI), jnp.bfloat16),
                pltpu.VMEM((DI, D), jnp.bfloat16),
            ],
        ),
        out_shape=jax.ShapeDtypeStruct((NPAD, D), jnp.float32),
        interpret=_INTERP,
    )(te_arr, x_disp, Wg, Wu, Wd)

    y0, y1 = _sc_combine(y, pos1, pos2)

    out = pl.pallas_call(
        _final_body,
        grid=(NQ,),
        in_specs=[
            pl.BlockSpec((BT, D), lambda i: (i, 0)),
            pl.BlockSpec((BT, D), lambda i: (i, 0)),
            pl.BlockSpec((BT, D), lambda i: (i, 0)),
            pl.BlockSpec((BT, E), lambda i: (i, 0)),
        ],
        out_specs=pl.BlockSpec((BT, D), lambda i: (i, 0)),
        out_shape=jax.ShapeDtypeStruct((S, D), jnp.float32),
        interpret=_INTERP,
    )(x, y0, y1, mf)

    return out.reshape(B, S, D)
